# Initial kernel scaffold; baseline (speedup 1.0000x reference)
import jax
import jax.numpy as jnp
from jax.experimental import pallas as pl


def kernel(p, x):
    raise NotImplementedError("write your pallas kernel here")


# speedup vs baseline: 58.4809x; 58.4809x over previous
import functools

import jax
import jax.numpy as jnp
from jax import lax
from jax.experimental import pallas as pl
from jax.experimental.pallas import tpu as pltpu
from jax.experimental.pallas import tpu_sc as plsc

B = 4
N = 16384
C = 64
M = 1024
K = 32

NUM_WORKERS = 32
M_PER_W = (B * M) // NUM_WORKERS
CH_PER_W = (B * C) // NUM_WORKERS

F_BIG = 1e30
I_BIG = 1 << 20


def _rne_bf16(x):
    b = lax.bitcast_convert_type(x, jnp.uint32)
    lsb = (b >> 16) & jnp.uint32(1)
    r = b + jnp.uint32(0x7FFF) + lsb
    return lax.bitcast_convert_type(r & jnp.uint32(0xFFFF0000), jnp.float32)


def _fps_body(px_ref, py_ref, pz_ref,
              idx_ref, cx_ref, cy_ref, cz_ref,
              cxr_ref, cyr_ref, czr_ref, c2_ref,
              pxr_ref, pyr_ref, pzr_ref, p2_ref,
              dists_ref):
    px = px_ref[...]
    py = py_ref[...]
    pz = pz_ref[...]

    pxr_ref[...] = _rne_bf16(px)
    pyr_ref[...] = _rne_bf16(py)
    pzr_ref[...] = _rne_bf16(pz)
    p2_ref[...] = (px * px + py * py) + pz * pz

    iota = lax.broadcasted_iota(jnp.int32, (B, N), 1)
    lane128 = lax.broadcasted_iota(jnp.int32, (B, 128), 1)

    cx0 = px[:, 0:1]
    cy0 = py[:, 0:1]
    cz0 = pz[:, 0:1]
    dists_ref[...] = jnp.full((B, N), 1e10, jnp.float32)

    col0 = lane128 == 0
    zf = jnp.zeros((B, 128), jnp.float32)
    zi = jnp.zeros((B, 128), jnp.int32)
    bufs0 = (zi,
             jnp.where(col0, cx0, zf), jnp.where(col0, cy0, zf),
             jnp.where(col0, cz0, zf),
             jnp.where(col0, _rne_bf16(cx0), zf),
             jnp.where(col0, _rne_bf16(cy0), zf),
             jnp.where(col0, _rne_bf16(cz0), zf),
             jnp.where(col0, (cx0 * cx0 + cy0 * cy0) + cz0 * cz0, zf))

    def step(i, carry):
        cx, cy, cz, bidx, bcx, bcy, bcz, bcxr, bcyr, bczr, bc2 = carry
        d = ((px - cx) ** 2 + (py - cy) ** 2) + (pz - cz) ** 2
        dn = jnp.minimum(dists_ref[...], d)
        dists_ref[...] = dn
        mx = jnp.max(dn, axis=1, keepdims=True)
        nxt = jnp.min(jnp.where(dn == mx, iota, N), axis=1, keepdims=True)
        sel = iota == nxt
        ncx = jnp.max(jnp.where(sel, px, -1.0), axis=1, keepdims=True)
        ncy = jnp.max(jnp.where(sel, py, -1.0), axis=1, keepdims=True)
        ncz = jnp.max(jnp.where(sel, pz, -1.0), axis=1, keepdims=True)
        at = lane128 == (i % 128)
        bidx = jnp.where(at, nxt.astype(jnp.int32), bidx)
        bcx = jnp.where(at, ncx, bcx)
        bcy = jnp.where(at, ncy, bcy)
        bcz = jnp.where(at, ncz, bcz)
        bcxr = jnp.where(at, _rne_bf16(ncx), bcxr)
        bcyr = jnp.where(at, _rne_bf16(ncy), bcyr)
        bczr = jnp.where(at, _rne_bf16(ncz), bczr)
        bc2 = jnp.where(at, (ncx * ncx + ncy * ncy) + ncz * ncz, bc2)

        @pl.when(i % 128 == 127)
        def _flush():
            off = pl.multiple_of((i // 128) * 128, 128)
            idx_ref[:, pl.ds(off, 128)] = bidx
            cx_ref[:, pl.ds(off, 128)] = bcx
            cy_ref[:, pl.ds(off, 128)] = bcy
            cz_ref[:, pl.ds(off, 128)] = bcz
            cxr_ref[:, pl.ds(off, 128)] = bcxr
            cyr_ref[:, pl.ds(off, 128)] = bcyr
            czr_ref[:, pl.ds(off, 128)] = bczr
            c2_ref[:, pl.ds(off, 128)] = bc2

        return (ncx, ncy, ncz, bidx, bcx, bcy, bcz, bcxr, bcyr, bczr, bc2)

    lax.fori_loop(1, M, step, (cx0, cy0, cz0) + bufs0)


def _run_fps(px, py, pz):
    f32 = jnp.float32
    outs = [
        jax.ShapeDtypeStruct((B, M), jnp.int32),
        jax.ShapeDtypeStruct((B, M), f32),
        jax.ShapeDtypeStruct((B, M), f32),
        jax.ShapeDtypeStruct((B, M), f32),
        jax.ShapeDtypeStruct((B, M), f32),
        jax.ShapeDtypeStruct((B, M), f32),
        jax.ShapeDtypeStruct((B, M), f32),
        jax.ShapeDtypeStruct((B, M), f32),
        jax.ShapeDtypeStruct((B, N), f32),
        jax.ShapeDtypeStruct((B, N), f32),
        jax.ShapeDtypeStruct((B, N), f32),
        jax.ShapeDtypeStruct((B, N), f32),
    ]
    return pl.pallas_call(
        _fps_body,
        out_shape=outs,
        scratch_shapes=[pltpu.VMEM((B, N), f32)],
    )(px, py, pz)


NBLK = 64
VPB = 16


def _knn_body(pxr_hbm, pyr_hbm, pzr_hbm, p2_hbm,
              cxr_hbm, cyr_hbm, czr_hbm, c2_hbm,
              nidx_hbm,
              px_v, py_v, pz_v, p2_v, d2_v, lvl_v,
              cx_v, cy_v, cz_v, c2_v, res_v, sem):
    wid = lax.axis_index("s") * 2 + lax.axis_index("c")
    b = wid // 8
    u = wid % 8

    pltpu.sync_copy(pxr_hbm.at[pl.ds(b * N, N)], px_v)
    pltpu.sync_copy(pyr_hbm.at[pl.ds(b * N, N)], py_v)
    pltpu.sync_copy(pzr_hbm.at[pl.ds(b * N, N)], pz_v)
    pltpu.sync_copy(p2_hbm.at[pl.ds(b * N, N)], p2_v)
    cbase = b * M + u * M_PER_W
    pltpu.sync_copy(cxr_hbm.at[pl.ds(cbase, M_PER_W)], cx_v)
    pltpu.sync_copy(cyr_hbm.at[pl.ds(cbase, M_PER_W)], cy_v)
    pltpu.sync_copy(czr_hbm.at[pl.ds(cbase, M_PER_W)], cz_v)
    pltpu.sync_copy(c2_hbm.at[pl.ds(cbase, M_PER_W)], c2_v)

    ii = lax.broadcasted_iota(jnp.int32, (16,), 0)

    def per_center(m, _):
        mvec = jnp.full((16,), 0, jnp.int32) + m
        cxs = plsc.load_gather(cx_v, [mvec])
        cys = plsc.load_gather(cy_v, [mvec])
        czs = plsc.load_gather(cz_v, [mvec])
        c2s = plsc.load_gather(c2_v, [mvec])

        def per_group(g, _):
            def per_block(blk, lvlvec):
                jb = g * 16 + blk
                bmin = jnp.full((16,), F_BIG, jnp.float32)
                for k in range(VPB):
                    o = pl.multiple_of(jb * 256 + k * 16, 16)
                    pxv = px_v[pl.ds(o, 16)]
                    pyv = py_v[pl.ds(o, 16)]
                    pzv = pz_v[pl.ds(o, 16)]
                    dot = cxs * pxv + (cys * pyv + czs * pzv)
                    d2v = (c2s - 2.0 * dot) + p2_v[pl.ds(o, 16)]
                    d2_v[pl.ds(o, 16)] = d2v
                    bmin = jnp.minimum(bmin, d2v)
                s = jnp.min(bmin)
                return jnp.where(ii == blk, s, lvlvec)

            lvlvec = lax.fori_loop(0, 16, per_block,
                                   jnp.full((16,), F_BIG, jnp.float32))
            lvl_v[pl.ds(pl.multiple_of(g * 16, 16), 16)] = lvlvec
            return 0

        lax.fori_loop(0, 4, per_group, 0)

        l0o = pl.multiple_of(0, 16)

        def per_round(r, carry):
            w0, w1 = carry
            l0 = lvl_v[pl.ds(0, 16)]
            l1 = lvl_v[pl.ds(16, 16)]
            l2 = lvl_v[pl.ds(32, 16)]
            l3 = lvl_v[pl.ds(48, 16)]
            t = jnp.min(jnp.minimum(jnp.minimum(l0, l1),
                                    jnp.minimum(l2, l3)))
            j0 = jnp.where(l0 == t, ii, I_BIG)
            j1 = jnp.where(l1 == t, ii + 16, I_BIG)
            j2 = jnp.where(l2 == t, ii + 32, I_BIG)
            j3 = jnp.where(l3 == t, ii + 48, I_BIG)
            j = jnp.min(jnp.minimum(jnp.minimum(j0, j1),
                                    jnp.minimum(j2, j3)))
            base = j * 256
            pv = jnp.full((16,), I_BIG, jnp.int32)
            for k in range(VPB):
                o = pl.multiple_of(base + k * 16, 16)
                dv = d2_v[pl.ds(o, 16)]
                pv = jnp.minimum(pv, jnp.where(dv == t, ii + k * 16, I_BIG))
            pos = jnp.min(pv)
            n = base + pos
            vo = pl.multiple_of(base + (pos // 16) * 16, 16)
            lane = pos % 16
            dv = d2_v[pl.ds(vo, 16)]
            d2_v[pl.ds(vo, 16)] = jnp.where(ii == lane, F_BIG, dv)
            bmin = jnp.full((16,), F_BIG, jnp.float32)
            for k in range(VPB):
                o = pl.multiple_of(base + k * 16, 16)
                bmin = jnp.minimum(bmin, d2_v[pl.ds(o, 16)])
            s2 = jnp.min(bmin)
            lo = pl.multiple_of((j // 16) * 16, 16)
            lv = lvl_v[pl.ds(lo, 16)]
            lvl_v[pl.ds(lo, 16)] = jnp.where(ii == (j % 16), s2, lv)
            w0 = jnp.where(ii == r, n, w0)
            w1 = jnp.where(ii == r - 16, n, w1)
            return (w0, w1)

        zero16 = jnp.zeros((16,), jnp.int32)
        w0, w1 = lax.fori_loop(0, K, per_round, (zero16, zero16))
        ro = pl.multiple_of(m * K, 16)
        res_v[pl.ds(ro, 16)] = w0
        res_v[pl.ds(ro + 16, 16)] = w1
        return 0

    lax.fori_loop(0, M_PER_W, per_center, 0)
    pltpu.sync_copy(res_v, nidx_hbm.at[pl.ds(wid * (M_PER_W * K),
                                             M_PER_W * K)])


def _run_knn(pxr, pyr, pzr, p2, cxr, cyr, czr, c2):
    f32 = jnp.float32
    mesh = plsc.VectorSubcoreMesh(core_axis_name="c", subcore_axis_name="s")
    kn = functools.partial(
        pl.kernel,
        out_type=jax.ShapeDtypeStruct((B * M * K,), jnp.int32),
        mesh=mesh,
        compiler_params=pltpu.CompilerParams(needs_layout_passes=False),
        scratch_types=[
            pltpu.VMEM((N,), f32),
            pltpu.VMEM((N,), f32),
            pltpu.VMEM((N,), f32),
            pltpu.VMEM((N,), f32),
            pltpu.VMEM((N,), f32),
            pltpu.VMEM((NBLK,), f32),
            pltpu.VMEM((M_PER_W,), f32),
            pltpu.VMEM((M_PER_W,), f32),
            pltpu.VMEM((M_PER_W,), f32),
            pltpu.VMEM((M_PER_W,), f32),
            pltpu.VMEM((M_PER_W * K,), jnp.int32),
            pltpu.SemaphoreType.DMA,
        ],
    )(_knn_body)
    return kn(pxr.reshape(-1), pyr.reshape(-1), pzr.reshape(-1),
              p2.reshape(-1), cxr.reshape(-1), cyr.reshape(-1),
              czr.reshape(-1), c2.reshape(-1))


def _gather_body(xflat_hbm, pxf_hbm, pyf_hbm, pzf_hbm,
                 cxf_hbm, cyf_hbm, czf_hbm, idx_hbm, nidx_hbm,
                 fj_hbm, cxo_hbm, gp_hbm,
                 nidx_v, absidx_v, vals_v, idx1_v, cabs_v, cvals_v, cd_v,
                 sem):
    wid = lax.axis_index("s") * 2 + lax.axis_index("c")
    b = wid // 8
    u = wid % 8
    MK = M * K

    pltpu.sync_copy(nidx_hbm.at[pl.ds(b * MK, MK)], nidx_v)
    pltpu.sync_copy(idx_hbm.at[pl.ds(b * M, M)], idx1_v)

    def add_off(i, off):
        o = pl.multiple_of(i * 16, 16)
        absidx_v[pl.ds(o, 16)] = nidx_v[pl.ds(o, 16)] + off
        return off

    def cadd_off(i, off):
        o = pl.multiple_of(i * 16, 16)
        cabs_v[pl.ds(o, 16)] = idx1_v[pl.ds(o, 16)] + off
        return off

    for c8 in range(CH_PER_W):
        ch = b * C + u * CH_PER_W + c8
        off = ch * N
        lax.fori_loop(0, MK // 16, add_off, off)
        pltpu.async_copy(xflat_hbm.at[absidx_v], vals_v, sem).wait()
        pltpu.sync_copy(vals_v, fj_hbm.at[pl.ds(ch * MK, MK)])
        lax.fori_loop(0, M // 16, cadd_off, off)
        pltpu.async_copy(xflat_hbm.at[cabs_v], cvals_v, sem).wait()
        pltpu.sync_copy(cvals_v, cxo_hbm.at[pl.ds(ch * M, M)])

    def do_coord(plane_hbm, cent_hbm, d):
        lax.fori_loop(0, MK // 16, add_off, b * N)
        pltpu.async_copy(plane_hbm.at[absidx_v], vals_v, sem).wait()
        pltpu.sync_copy(cent_hbm.at[pl.ds(b * M, M)], cd_v)

        def sub(i, _):
            o = pl.multiple_of(i * 16, 16)
            gvec = jnp.full((16,), 0, jnp.int32) + i // 2
            cs = plsc.load_gather(cd_v, [gvec])
            vals_v[pl.ds(o, 16)] = vals_v[pl.ds(o, 16)] - cs
            return 0

        lax.fori_loop(0, MK // 16, sub, 0)
        pltpu.sync_copy(vals_v, gp_hbm.at[pl.ds((b * 3 + d) * MK, MK)])

    @pl.when(u == 0)
    def _():
        do_coord(pxf_hbm, cxf_hbm, 0)

    @pl.when(u == 1)
    def _():
        do_coord(pyf_hbm, cyf_hbm, 1)

    @pl.when(u == 2)
    def _():
        do_coord(pzf_hbm, czf_hbm, 2)


def _run_gather(x, px, py, pz, cx, cy, cz, idx, nidx):
    f32 = jnp.float32
    MK = M * K
    mesh = plsc.VectorSubcoreMesh(core_axis_name="c", subcore_axis_name="s")
    kn = functools.partial(
        pl.kernel,
        out_type=(
            jax.ShapeDtypeStruct((B * C * MK,), f32),
            jax.ShapeDtypeStruct((B * C * M,), f32),
            jax.ShapeDtypeStruct((B * 3 * MK,), f32),
        ),
        mesh=mesh,
        compiler_params=pltpu.CompilerParams(needs_layout_passes=False),
        scratch_types=[
            pltpu.VMEM((MK,), jnp.int32),
            pltpu.VMEM((MK,), jnp.int32),
            pltpu.VMEM((MK,), f32),
            pltpu.VMEM((M,), jnp.int32),
            pltpu.VMEM((M,), jnp.int32),
            pltpu.VMEM((M,), f32),
            pltpu.VMEM((M,), f32),
            pltpu.SemaphoreType.DMA,
        ],
    )(_gather_body)
    return kn(x.reshape(-1), px.reshape(-1), py.reshape(-1), pz.reshape(-1),
              cx.reshape(-1), cy.reshape(-1), cz.reshape(-1),
              idx.reshape(-1), nidx)


def kernel(p, x):
    px = p[:, :, 0]
    py = p[:, :, 1]
    pz = p[:, :, 2]

    (idx, cx, cy, cz, cxr, cyr, czr, c2,
     pxr, pyr, pzr, p2) = _run_fps(px, py, pz)

    nidx = _run_knn(pxr, pyr, pzr, p2, cxr, cyr, czr, c2)

    fjf, cxof, gpf = _run_gather(x, px, py, pz, cx, cy, cz, idx, nidx)

    grouped_p = gpf.reshape(B, 3, M, K)
    center_p = jnp.stack([cx, cy, cz], axis=-1)
    fj = fjf.reshape(B, C, M, K)
    center_x = cxof.reshape(B, C, M, 1)
    return (grouped_p, center_p, fj, center_x)


# speedup vs baseline: 75.9004x; 1.2979x over previous
import functools

import jax
import jax.numpy as jnp
from jax import lax
from jax.experimental import pallas as pl
from jax.experimental.pallas import tpu as pltpu
from jax.experimental.pallas import tpu_sc as plsc

B = 4
N = 16384
C = 64
M = 1024
K = 32

NUM_WORKERS = 32
M_PER_W = (B * M) // NUM_WORKERS
CH_PER_W = (B * C) // NUM_WORKERS

F_BIG = 1e30
I_BIG = 1 << 20


def _rne_bf16(x):
    b = lax.bitcast_convert_type(x, jnp.uint32)
    lsb = (b >> 16) & jnp.uint32(1)
    r = b + jnp.uint32(0x7FFF) + lsb
    return lax.bitcast_convert_type(r & jnp.uint32(0xFFFF0000), jnp.float32)


def _fps_body(px_ref, py_ref, pz_ref,
              idx_ref, cx_ref, cy_ref, cz_ref,
              cxr_ref, cyr_ref, czr_ref, c2_ref,
              pxr_ref, pyr_ref, pzr_ref, p2_ref,
              dists_ref):
    px = px_ref[...]
    py = py_ref[...]
    pz = pz_ref[...]

    pxr_ref[...] = _rne_bf16(px)
    pyr_ref[...] = _rne_bf16(py)
    pzr_ref[...] = _rne_bf16(pz)
    p2_ref[...] = (px * px + py * py) + pz * pz

    iota = lax.broadcasted_iota(jnp.int32, (B, N), 1)
    lane128 = lax.broadcasted_iota(jnp.int32, (B, 128), 1)

    cx0 = px[:, 0:1]
    cy0 = py[:, 0:1]
    cz0 = pz[:, 0:1]
    dists_ref[...] = jnp.full((B, N), 1e10, jnp.float32)

    col0 = lane128 == 0
    zf = jnp.zeros((B, 128), jnp.float32)
    zi = jnp.zeros((B, 128), jnp.int32)
    bufs0 = (zi,
             jnp.where(col0, cx0, zf), jnp.where(col0, cy0, zf),
             jnp.where(col0, cz0, zf),
             jnp.where(col0, _rne_bf16(cx0), zf),
             jnp.where(col0, _rne_bf16(cy0), zf),
             jnp.where(col0, _rne_bf16(cz0), zf),
             jnp.where(col0, (cx0 * cx0 + cy0 * cy0) + cz0 * cz0, zf))

    def step(i, carry):
        cx, cy, cz, bidx, bcx, bcy, bcz, bcxr, bcyr, bczr, bc2 = carry
        d = ((px - cx) ** 2 + (py - cy) ** 2) + (pz - cz) ** 2
        dn = jnp.minimum(dists_ref[...], d)
        dists_ref[...] = dn
        mx = jnp.max(dn, axis=1, keepdims=True)
        nxt = jnp.min(jnp.where(dn == mx, iota, N), axis=1, keepdims=True)
        sel = iota == nxt
        ncx = jnp.max(jnp.where(sel, px, -1.0), axis=1, keepdims=True)
        ncy = jnp.max(jnp.where(sel, py, -1.0), axis=1, keepdims=True)
        ncz = jnp.max(jnp.where(sel, pz, -1.0), axis=1, keepdims=True)
        at = lane128 == (i % 128)
        bidx = jnp.where(at, nxt.astype(jnp.int32), bidx)
        bcx = jnp.where(at, ncx, bcx)
        bcy = jnp.where(at, ncy, bcy)
        bcz = jnp.where(at, ncz, bcz)
        bcxr = jnp.where(at, _rne_bf16(ncx), bcxr)
        bcyr = jnp.where(at, _rne_bf16(ncy), bcyr)
        bczr = jnp.where(at, _rne_bf16(ncz), bczr)
        bc2 = jnp.where(at, (ncx * ncx + ncy * ncy) + ncz * ncz, bc2)

        @pl.when(i % 128 == 127)
        def _flush():
            off = pl.multiple_of((i // 128) * 128, 128)
            idx_ref[:, pl.ds(off, 128)] = bidx
            cx_ref[:, pl.ds(off, 128)] = bcx
            cy_ref[:, pl.ds(off, 128)] = bcy
            cz_ref[:, pl.ds(off, 128)] = bcz
            cxr_ref[:, pl.ds(off, 128)] = bcxr
            cyr_ref[:, pl.ds(off, 128)] = bcyr
            czr_ref[:, pl.ds(off, 128)] = bczr
            c2_ref[:, pl.ds(off, 128)] = bc2

        return (ncx, ncy, ncz, bidx, bcx, bcy, bcz, bcxr, bcyr, bczr, bc2)

    lax.fori_loop(1, M, step, (cx0, cy0, cz0) + bufs0)


def _run_fps(px, py, pz):
    f32 = jnp.float32
    outs = [
        jax.ShapeDtypeStruct((B, M), jnp.int32),
        jax.ShapeDtypeStruct((B, M), f32),
        jax.ShapeDtypeStruct((B, M), f32),
        jax.ShapeDtypeStruct((B, M), f32),
        jax.ShapeDtypeStruct((B, M), f32),
        jax.ShapeDtypeStruct((B, M), f32),
        jax.ShapeDtypeStruct((B, M), f32),
        jax.ShapeDtypeStruct((B, M), f32),
        jax.ShapeDtypeStruct((B, N), f32),
        jax.ShapeDtypeStruct((B, N), f32),
        jax.ShapeDtypeStruct((B, N), f32),
        jax.ShapeDtypeStruct((B, N), f32),
    ]
    return pl.pallas_call(
        _fps_body,
        out_shape=outs,
        scratch_shapes=[pltpu.VMEM((B, N), f32)],
    )(px, py, pz)


NBLK = 64
VPB = 16


def _knn_body(pxr_hbm, pyr_hbm, pzr_hbm, p2_hbm,
              cxr_hbm, cyr_hbm, czr_hbm, c2_hbm,
              nidx_hbm,
              px_v, py_v, pz_v, p2_v, d2_v, d2b_v,
              cx_v, cy_v, cz_v, c2_v, res_v, sem):
    wid = lax.axis_index("s") * 2 + lax.axis_index("c")
    b = wid // 8
    u = wid % 8

    pltpu.sync_copy(pxr_hbm.at[pl.ds(b * N, N)], px_v)
    pltpu.sync_copy(pyr_hbm.at[pl.ds(b * N, N)], py_v)
    pltpu.sync_copy(pzr_hbm.at[pl.ds(b * N, N)], pz_v)
    pltpu.sync_copy(p2_hbm.at[pl.ds(b * N, N)], p2_v)
    cbase = b * M + u * M_PER_W
    pltpu.sync_copy(cxr_hbm.at[pl.ds(cbase, M_PER_W)], cx_v)
    pltpu.sync_copy(cyr_hbm.at[pl.ds(cbase, M_PER_W)], cy_v)
    pltpu.sync_copy(czr_hbm.at[pl.ds(cbase, M_PER_W)], cz_v)
    pltpu.sync_copy(c2_hbm.at[pl.ds(cbase, M_PER_W)], c2_v)

    ii = lax.broadcasted_iota(jnp.int32, (16,), 0)

    def select32(d2ref, lvl, m):
        def per_round(r, carry):
            w0, w1, l0, l1, l2, l3 = carry
            t = jnp.min(jnp.minimum(jnp.minimum(l0, l1),
                                    jnp.minimum(l2, l3)))
            j0 = jnp.where(l0 == t, ii, I_BIG)
            j1 = jnp.where(l1 == t, ii + 16, I_BIG)
            j2 = jnp.where(l2 == t, ii + 32, I_BIG)
            j3 = jnp.where(l3 == t, ii + 48, I_BIG)
            j = jnp.min(jnp.minimum(jnp.minimum(j0, j1),
                                    jnp.minimum(j2, j3)))
            base = j * 256
            pv = jnp.full((16,), I_BIG, jnp.int32)
            for k in range(VPB):
                o = pl.multiple_of(base + k * 16, 16)
                dv = d2ref[pl.ds(o, 16)]
                pv = jnp.minimum(pv, jnp.where(dv == t, ii + k * 16, I_BIG))
            pos = jnp.min(pv)
            n = base + pos
            vo = pl.multiple_of(base + (pos // 16) * 16, 16)
            lane = pos % 16
            dv = d2ref[pl.ds(vo, 16)]
            d2ref[pl.ds(vo, 16)] = jnp.where(ii == lane, F_BIG, dv)
            bmin = jnp.full((16,), F_BIG, jnp.float32)
            for k in range(VPB):
                o = pl.multiple_of(base + k * 16, 16)
                bmin = jnp.minimum(bmin, d2ref[pl.ds(o, 16)])
            s2 = jnp.min(bmin)
            gsel = j // 16
            lsel = ii == (j % 16)
            l0 = jnp.where(lsel & (gsel == 0), s2, l0)
            l1 = jnp.where(lsel & (gsel == 1), s2, l1)
            l2 = jnp.where(lsel & (gsel == 2), s2, l2)
            l3 = jnp.where(lsel & (gsel == 3), s2, l3)
            w0 = jnp.where(ii == r, n, w0)
            w1 = jnp.where(ii == r - 16, n, w1)
            return (w0, w1, l0, l1, l2, l3)

        zero16 = jnp.zeros((16,), jnp.int32)
        w0, w1 = lax.fori_loop(0, K, per_round,
                               (zero16, zero16) + lvl)[:2]
        ro = pl.multiple_of(m * K, 16)
        res_v[pl.ds(ro, 16)] = w0
        res_v[pl.ds(ro + 16, 16)] = w1

    def per_pair(h, _):
        ma = 2 * h
        mb = 2 * h + 1
        mav = jnp.full((16,), 0, jnp.int32) + ma
        mbv = jnp.full((16,), 0, jnp.int32) + mb
        axs = plsc.load_gather(cx_v, [mav]) * -2.0
        ays = plsc.load_gather(cy_v, [mav]) * -2.0
        azs = plsc.load_gather(cz_v, [mav]) * -2.0
        a2s = plsc.load_gather(c2_v, [mav])
        bxs = plsc.load_gather(cx_v, [mbv]) * -2.0
        bys = plsc.load_gather(cy_v, [mbv]) * -2.0
        bzs = plsc.load_gather(cz_v, [mbv]) * -2.0
        b2s = plsc.load_gather(c2_v, [mbv])

        lvla = []
        lvlb = []
        for g in range(4):
            def per_block(blk, carry):
                la, lb = carry
                jb = g * 16 + blk
                bma = jnp.full((16,), F_BIG, jnp.float32)
                bmb = jnp.full((16,), F_BIG, jnp.float32)
                for k in range(VPB):
                    o = pl.multiple_of(jb * 256 + k * 16, 16)
                    pxv = px_v[pl.ds(o, 16)]
                    pyv = py_v[pl.ds(o, 16)]
                    pzv = pz_v[pl.ds(o, 16)]
                    p2v = p2_v[pl.ds(o, 16)]
                    da = (a2s + (axs * pxv + (ays * pyv + azs * pzv))) + p2v
                    db = (b2s + (bxs * pxv + (bys * pyv + bzs * pzv))) + p2v
                    d2_v[pl.ds(o, 16)] = da
                    d2b_v[pl.ds(o, 16)] = db
                    bma = jnp.minimum(bma, da)
                    bmb = jnp.minimum(bmb, db)
                sa = jnp.min(bma)
                sb = jnp.min(bmb)
                la = jnp.where(ii == blk, sa, la)
                lb = jnp.where(ii == blk, sb, lb)
                return (la, lb)

            fb = jnp.full((16,), F_BIG, jnp.float32)
            la, lb = lax.fori_loop(0, 16, per_block, (fb, fb))
            lvla.append(la)
            lvlb.append(lb)

        select32(d2_v, tuple(lvla), ma)
        select32(d2b_v, tuple(lvlb), mb)
        return 0

    lax.fori_loop(0, M_PER_W // 2, per_pair, 0)
    pltpu.sync_copy(res_v, nidx_hbm.at[pl.ds(wid * (M_PER_W * K),
                                             M_PER_W * K)])


def _run_knn(pxr, pyr, pzr, p2, cxr, cyr, czr, c2):
    f32 = jnp.float32
    mesh = plsc.VectorSubcoreMesh(core_axis_name="c", subcore_axis_name="s")
    kn = functools.partial(
        pl.kernel,
        out_type=jax.ShapeDtypeStruct((B * M * K,), jnp.int32),
        mesh=mesh,
        compiler_params=pltpu.CompilerParams(needs_layout_passes=False),
        scratch_types=[
            pltpu.VMEM((N,), f32),
            pltpu.VMEM((N,), f32),
            pltpu.VMEM((N,), f32),
            pltpu.VMEM((N,), f32),
            pltpu.VMEM((N,), f32),
            pltpu.VMEM((N,), f32),
            pltpu.VMEM((M_PER_W,), f32),
            pltpu.VMEM((M_PER_W,), f32),
            pltpu.VMEM((M_PER_W,), f32),
            pltpu.VMEM((M_PER_W,), f32),
            pltpu.VMEM((M_PER_W * K,), jnp.int32),
            pltpu.SemaphoreType.DMA,
        ],
    )(_knn_body)
    return kn(pxr.reshape(-1), pyr.reshape(-1), pzr.reshape(-1),
              p2.reshape(-1), cxr.reshape(-1), cyr.reshape(-1),
              czr.reshape(-1), c2.reshape(-1))


def _gather_body(xflat_hbm, pxf_hbm, pyf_hbm, pzf_hbm,
                 cxf_hbm, cyf_hbm, czf_hbm, idx_hbm, nidx_hbm,
                 fj_hbm, cxo_hbm, gp_hbm,
                 nidx_v, absidx_v, vals_v, idx1_v, cabs_v, cvals_v, cd_v,
                 sem):
    wid = lax.axis_index("s") * 2 + lax.axis_index("c")
    b = wid // 8
    u = wid % 8
    MK = M * K

    pltpu.sync_copy(nidx_hbm.at[pl.ds(b * MK, MK)], nidx_v)
    pltpu.sync_copy(idx_hbm.at[pl.ds(b * M, M)], idx1_v)

    def add_off(i, off):
        o = pl.multiple_of(i * 16, 16)
        absidx_v[pl.ds(o, 16)] = nidx_v[pl.ds(o, 16)] + off
        return off

    def cadd_off(i, off):
        o = pl.multiple_of(i * 16, 16)
        cabs_v[pl.ds(o, 16)] = idx1_v[pl.ds(o, 16)] + off
        return off

    for c8 in range(CH_PER_W):
        ch = b * C + u * CH_PER_W + c8
        off = ch * N
        lax.fori_loop(0, MK // 16, add_off, off)
        pltpu.async_copy(xflat_hbm.at[absidx_v], vals_v, sem).wait()
        pltpu.sync_copy(vals_v, fj_hbm.at[pl.ds(ch * MK, MK)])
        lax.fori_loop(0, M // 16, cadd_off, off)
        pltpu.async_copy(xflat_hbm.at[cabs_v], cvals_v, sem).wait()
        pltpu.sync_copy(cvals_v, cxo_hbm.at[pl.ds(ch * M, M)])

    def do_coord(plane_hbm, cent_hbm, d):
        lax.fori_loop(0, MK // 16, add_off, b * N)
        pltpu.async_copy(plane_hbm.at[absidx_v], vals_v, sem).wait()
        pltpu.sync_copy(cent_hbm.at[pl.ds(b * M, M)], cd_v)

        def sub(i, _):
            o = pl.multiple_of(i * 16, 16)
            gvec = jnp.full((16,), 0, jnp.int32) + i // 2
            cs = plsc.load_gather(cd_v, [gvec])
            vals_v[pl.ds(o, 16)] = vals_v[pl.ds(o, 16)] - cs
            return 0

        lax.fori_loop(0, MK // 16, sub, 0)
        pltpu.sync_copy(vals_v, gp_hbm.at[pl.ds((b * 3 + d) * MK, MK)])

    @pl.when(u == 0)
    def _():
        do_coord(pxf_hbm, cxf_hbm, 0)

    @pl.when(u == 1)
    def _():
        do_coord(pyf_hbm, cyf_hbm, 1)

    @pl.when(u == 2)
    def _():
        do_coord(pzf_hbm, czf_hbm, 2)


def _run_gather(x, px, py, pz, cx, cy, cz, idx, nidx):
    f32 = jnp.float32
    MK = M * K
    mesh = plsc.VectorSubcoreMesh(core_axis_name="c", subcore_axis_name="s")
    kn = functools.partial(
        pl.kernel,
        out_type=(
            jax.ShapeDtypeStruct((B * C * MK,), f32),
            jax.ShapeDtypeStruct((B * C * M,), f32),
            jax.ShapeDtypeStruct((B * 3 * MK,), f32),
        ),
        mesh=mesh,
        compiler_params=pltpu.CompilerParams(needs_layout_passes=False),
        scratch_types=[
            pltpu.VMEM((MK,), jnp.int32),
            pltpu.VMEM((MK,), jnp.int32),
            pltpu.VMEM((MK,), f32),
            pltpu.VMEM((M,), jnp.int32),
            pltpu.VMEM((M,), jnp.int32),
            pltpu.VMEM((M,), f32),
            pltpu.VMEM((M,), f32),
            pltpu.SemaphoreType.DMA,
        ],
    )(_gather_body)
    return kn(x.reshape(-1), px.reshape(-1), py.reshape(-1), pz.reshape(-1),
              cx.reshape(-1), cy.reshape(-1), cz.reshape(-1),
              idx.reshape(-1), nidx)


def kernel(p, x):
    px = p[:, :, 0]
    py = p[:, :, 1]
    pz = p[:, :, 2]

    (idx, cx, cy, cz, cxr, cyr, czr, c2,
     pxr, pyr, pzr, p2) = _run_fps(px, py, pz)

    nidx = _run_knn(pxr, pyr, pzr, p2, cxr, cyr, czr, c2)

    fjf, cxof, gpf = _run_gather(x, px, py, pz, cx, cy, cz, idx, nidx)

    grouped_p = gpf.reshape(B, 3, M, K)
    center_p = jnp.stack([cx, cy, cz], axis=-1)
    fj = fjf.reshape(B, C, M, K)
    center_x = cxof.reshape(B, C, M, 1)
    return (grouped_p, center_p, fj, center_x)


# speedup vs baseline: 91.7442x; 1.2087x over previous
import functools

import jax
import jax.numpy as jnp
from jax import lax
from jax.experimental import pallas as pl
from jax.experimental.pallas import tpu as pltpu
from jax.experimental.pallas import tpu_sc as plsc

B = 4
N = 16384
C = 64
M = 1024
K = 32

NUM_WORKERS = 32
M_PER_W = (B * M) // NUM_WORKERS
CH_PER_W = (B * C) // NUM_WORKERS

F_BIG = 1e30
I_BIG = 1 << 20


def _rne_bf16(x):
    b = lax.bitcast_convert_type(x, jnp.uint32)
    lsb = (b >> 16) & jnp.uint32(1)
    r = b + jnp.uint32(0x7FFF) + lsb
    return lax.bitcast_convert_type(r & jnp.uint32(0xFFFF0000), jnp.float32)


SL = 8 * B
NL = N // 8


def _fps_body(px_ref, py_ref, pz_ref,
              idx_ref, cx_ref, cy_ref, cz_ref, c2_ref, p2_ref,
              dists_ref):
    px = px_ref[...]
    py = py_ref[...]
    pz = pz_ref[...]

    p2_ref[...] = (px * px + py * py) + pz * pz

    iota = (lax.broadcasted_iota(jnp.int32, (SL, NL), 0) % 8) * NL \
        + lax.broadcasted_iota(jnp.int32, (SL, NL), 1)
    lane128 = lax.broadcasted_iota(jnp.int32, (B, 128), 1)

    dists_ref[...] = jnp.full((SL, NL), 1e10, jnp.float32)

    def brow(scalars):
        return jnp.concatenate([jnp.full((8, 1), s) for s in scalars], axis=0)

    def bcol(scalars, dtype):
        return jnp.concatenate(
            [jnp.full((1, 1), s, dtype) for s in scalars], axis=0)

    cx0s = [px[8 * b, 0] for b in range(B)]
    cy0s = [py[8 * b, 0] for b in range(B)]
    cz0s = [pz[8 * b, 0] for b in range(B)]

    col0 = lane128 == 0
    zf = jnp.zeros((B, 128), jnp.float32)
    zi = jnp.zeros((B, 128), jnp.int32)
    ccx0 = bcol(cx0s, jnp.float32)
    ccy0 = bcol(cy0s, jnp.float32)
    ccz0 = bcol(cz0s, jnp.float32)
    bufs0 = (zi,
             jnp.where(col0, ccx0, zf), jnp.where(col0, ccy0, zf),
             jnp.where(col0, ccz0, zf),
             jnp.where(col0, (ccx0 * ccx0 + ccy0 * ccy0) + ccz0 * ccz0, zf))

    def step(i, carry):
        cx, cy, cz, bidx, bcx, bcy, bcz, bc2 = carry
        d = ((px - cx) ** 2 + (py - cy) ** 2) + (pz - cz) ** 2
        dn = jnp.minimum(dists_ref[...], d)
        dists_ref[...] = dn
        m1 = jnp.max(dn, axis=1, keepdims=True)
        mxs = [jnp.max(m1[8 * b:8 * b + 8, 0:1]) for b in range(B)]
        eq = jnp.where(dn == brow(mxs), iota, N)
        n1 = jnp.min(eq, axis=1, keepdims=True)
        nxts = [jnp.min(n1[8 * b:8 * b + 8, 0:1]) for b in range(B)]
        sel = iota == brow(nxts)
        gx = jnp.max(jnp.where(sel, px, -1.0), axis=1, keepdims=True)
        gy = jnp.max(jnp.where(sel, py, -1.0), axis=1, keepdims=True)
        gz = jnp.max(jnp.where(sel, pz, -1.0), axis=1, keepdims=True)
        cxs = [jnp.max(gx[8 * b:8 * b + 8, 0:1]) for b in range(B)]
        cys = [jnp.max(gy[8 * b:8 * b + 8, 0:1]) for b in range(B)]
        czs = [jnp.max(gz[8 * b:8 * b + 8, 0:1]) for b in range(B)]
        ncx = bcol(cxs, jnp.float32)
        ncy = bcol(cys, jnp.float32)
        ncz = bcol(czs, jnp.float32)
        nxt = bcol(nxts, jnp.int32)
        at = lane128 == (i % 128)
        bidx = jnp.where(at, nxt, bidx)
        bcx = jnp.where(at, ncx, bcx)
        bcy = jnp.where(at, ncy, bcy)
        bcz = jnp.where(at, ncz, bcz)
        bc2 = jnp.where(at, (ncx * ncx + ncy * ncy) + ncz * ncz, bc2)

        @pl.when(i % 128 == 127)
        def _flush():
            off = pl.multiple_of((i // 128) * 128, 128)
            idx_ref[:, pl.ds(off, 128)] = bidx
            cx_ref[:, pl.ds(off, 128)] = bcx
            cy_ref[:, pl.ds(off, 128)] = bcy
            cz_ref[:, pl.ds(off, 128)] = bcz
            c2_ref[:, pl.ds(off, 128)] = bc2

        return (brow(cxs), brow(cys), brow(czs),
                bidx, bcx, bcy, bcz, bc2)

    lax.fori_loop(1, M, step, (brow(cx0s), brow(cy0s), brow(cz0s)) + bufs0)


def _run_fps(px, py, pz):
    f32 = jnp.float32
    outs = [
        jax.ShapeDtypeStruct((B, M), jnp.int32),
        jax.ShapeDtypeStruct((B, M), f32),
        jax.ShapeDtypeStruct((B, M), f32),
        jax.ShapeDtypeStruct((B, M), f32),
        jax.ShapeDtypeStruct((B, M), f32),
        jax.ShapeDtypeStruct((SL, NL), f32),
    ]
    return pl.pallas_call(
        _fps_body,
        out_shape=outs,
        scratch_shapes=[pltpu.VMEM((SL, NL), f32)],
    )(px.reshape(SL, NL), py.reshape(SL, NL), pz.reshape(SL, NL))


NBLK = 64
VPB = 16


MBLK = 128


def _d2_body(cp_ref, pt_ref, c2_ref, p2_ref, out_ref):
    cp = cp_ref[0]
    pt = pt_ref[0]
    ein = jnp.dot(cp, pt, preferred_element_type=jnp.float32)
    out_ref[0] = (c2_ref[0] - 2.0 * ein) + p2_ref[0]


def _run_d2(center_p, p_t, c2c, p2r):
    return pl.pallas_call(
        _d2_body,
        grid=(B, M // MBLK),
        in_specs=[
            pl.BlockSpec((1, MBLK, 3), lambda b, m: (b, m, 0)),
            pl.BlockSpec((1, 3, N), lambda b, m: (b, 0, 0)),
            pl.BlockSpec((1, MBLK, 1), lambda b, m: (b, m, 0)),
            pl.BlockSpec((1, 1, N), lambda b, m: (b, 0, 0)),
        ],
        out_specs=pl.BlockSpec((1, MBLK, N), lambda b, m: (b, m, 0)),
        out_shape=jax.ShapeDtypeStruct((B, M, N), jnp.float32),
    )(center_p, p_t, c2c, p2r)


def _sel_body(d2_hbm, nidx_hbm, rowa_v, rowb_v, res_v, sema, semb):
    wid = lax.axis_index("s") * 2 + lax.axis_index("c")
    wbase = wid * M_PER_W

    ii = lax.broadcasted_iota(jnp.int32, (16,), 0)

    def select32(d2ref, lvl, m):
        def per_round(r, carry):
            w0, w1, l0, l1, l2, l3 = carry
            t = jnp.min(jnp.minimum(jnp.minimum(l0, l1),
                                    jnp.minimum(l2, l3)))
            j0 = jnp.where(l0 == t, ii, I_BIG)
            j1 = jnp.where(l1 == t, ii + 16, I_BIG)
            j2 = jnp.where(l2 == t, ii + 32, I_BIG)
            j3 = jnp.where(l3 == t, ii + 48, I_BIG)
            j = jnp.min(jnp.minimum(jnp.minimum(j0, j1),
                                    jnp.minimum(j2, j3)))
            base = j * 256
            pv = jnp.full((16,), I_BIG, jnp.int32)
            for k in range(VPB):
                o = pl.multiple_of(base + k * 16, 16)
                dv = d2ref[pl.ds(o, 16)]
                pv = jnp.minimum(pv, jnp.where(dv == t, ii + k * 16, I_BIG))
            pos = jnp.min(pv)
            n = base + pos
            vo = pl.multiple_of(base + (pos // 16) * 16, 16)
            lane = pos % 16
            dv = d2ref[pl.ds(vo, 16)]
            d2ref[pl.ds(vo, 16)] = jnp.where(ii == lane, F_BIG, dv)
            bmin = jnp.full((16,), F_BIG, jnp.float32)
            for k in range(VPB):
                o = pl.multiple_of(base + k * 16, 16)
                bmin = jnp.minimum(bmin, d2ref[pl.ds(o, 16)])
            s2 = jnp.min(bmin)
            gsel = j // 16
            lsel = ii == (j % 16)
            l0 = jnp.where(lsel & (gsel == 0), s2, l0)
            l1 = jnp.where(lsel & (gsel == 1), s2, l1)
            l2 = jnp.where(lsel & (gsel == 2), s2, l2)
            l3 = jnp.where(lsel & (gsel == 3), s2, l3)
            w0 = jnp.where(ii == r, n, w0)
            w1 = jnp.where(ii == r - 16, n, w1)
            return (w0, w1, l0, l1, l2, l3)

        zero16 = jnp.zeros((16,), jnp.int32)
        w0, w1 = lax.fori_loop(0, K, per_round,
                               (zero16, zero16) + lvl)[:2]
        ro = pl.multiple_of(m * K, 16)
        res_v[pl.ds(ro, 16)] = w0
        res_v[pl.ds(ro + 16, 16)] = w1

    def lvlpass(rowref):
        lvl = []
        for g in range(4):
            def per_block(blk, la):
                jb = g * 16 + blk
                bma = jnp.full((16,), F_BIG, jnp.float32)
                for k in range(VPB):
                    o = pl.multiple_of(jb * 256 + k * 16, 16)
                    bma = jnp.minimum(bma, rowref[pl.ds(o, 16)])
                return jnp.where(ii == blk, jnp.min(bma), la)

            lvl.append(lax.fori_loop(
                0, 16, per_block, jnp.full((16,), F_BIG, jnp.float32)))
        return tuple(lvl)

    pltpu.async_copy(d2_hbm.at[pl.ds(wbase * N, N)], rowa_v, sema)

    def per_pair(h, _):
        mca = 2 * h
        mcb = 2 * h + 1
        hb = pltpu.async_copy(
            d2_hbm.at[pl.ds((wbase + mcb) * N, N)], rowb_v, semb)
        pltpu.make_async_copy(
            d2_hbm.at[pl.ds(0, N)], rowa_v, sema).wait()
        select32(rowa_v, lvlpass(rowa_v), mca)
        nxt = jnp.where(mca + 2 < M_PER_W, mca + 2, 0)
        pltpu.async_copy(
            d2_hbm.at[pl.ds((wbase + nxt) * N, N)], rowa_v, sema)
        hb.wait()
        select32(rowb_v, lvlpass(rowb_v), mcb)
        return 0

    lax.fori_loop(0, M_PER_W // 2, per_pair, 0)
    pltpu.make_async_copy(d2_hbm.at[pl.ds(0, N)], rowa_v, sema).wait()
    pltpu.sync_copy(res_v, nidx_hbm.at[pl.ds(wid * (M_PER_W * K),
                                             M_PER_W * K)])


def _run_sel(d2):
    f32 = jnp.float32
    mesh = plsc.VectorSubcoreMesh(core_axis_name="c", subcore_axis_name="s")
    kn = functools.partial(
        pl.kernel,
        out_type=jax.ShapeDtypeStruct((B * M * K,), jnp.int32),
        mesh=mesh,
        compiler_params=pltpu.CompilerParams(needs_layout_passes=False),
        scratch_types=[
            pltpu.VMEM((N,), f32),
            pltpu.VMEM((N,), f32),
            pltpu.VMEM((M_PER_W * K,), jnp.int32),
            pltpu.SemaphoreType.DMA,
            pltpu.SemaphoreType.DMA,
        ],
    )(_sel_body)
    return kn(d2.reshape(-1))


def _gather_body(xflat_hbm, pxf_hbm, pyf_hbm, pzf_hbm,
                 cxf_hbm, cyf_hbm, czf_hbm, idx_hbm, nidx_hbm,
                 fj_hbm, cxo_hbm, gp_hbm,
                 nidx_v, absidx_v, vals_v, idx1_v, cabs_v, cvals_v, cd_v,
                 sem):
    wid = lax.axis_index("s") * 2 + lax.axis_index("c")
    b = wid // 8
    u = wid % 8
    MK = M * K

    pltpu.sync_copy(nidx_hbm.at[pl.ds(b * MK, MK)], nidx_v)
    pltpu.sync_copy(idx_hbm.at[pl.ds(b * M, M)], idx1_v)

    def add_off(i, off):
        o = pl.multiple_of(i * 16, 16)
        absidx_v[pl.ds(o, 16)] = nidx_v[pl.ds(o, 16)] + off
        return off

    def cadd_off(i, off):
        o = pl.multiple_of(i * 16, 16)
        cabs_v[pl.ds(o, 16)] = idx1_v[pl.ds(o, 16)] + off
        return off

    for c8 in range(CH_PER_W):
        ch = b * C + u * CH_PER_W + c8
        off = ch * N
        lax.fori_loop(0, MK // 16, add_off, off)
        pltpu.async_copy(xflat_hbm.at[absidx_v], vals_v, sem).wait()
        pltpu.sync_copy(vals_v, fj_hbm.at[pl.ds(ch * MK, MK)])
        lax.fori_loop(0, M // 16, cadd_off, off)
        pltpu.async_copy(xflat_hbm.at[cabs_v], cvals_v, sem).wait()
        pltpu.sync_copy(cvals_v, cxo_hbm.at[pl.ds(ch * M, M)])

    def do_coord(plane_hbm, cent_hbm, d):
        lax.fori_loop(0, MK // 16, add_off, b * N)
        pltpu.async_copy(plane_hbm.at[absidx_v], vals_v, sem).wait()
        pltpu.sync_copy(cent_hbm.at[pl.ds(b * M, M)], cd_v)

        def sub(i, _):
            o = pl.multiple_of(i * 16, 16)
            gvec = jnp.full((16,), 0, jnp.int32) + i // 2
            cs = plsc.load_gather(cd_v, [gvec])
            vals_v[pl.ds(o, 16)] = vals_v[pl.ds(o, 16)] - cs
            return 0

        lax.fori_loop(0, MK // 16, sub, 0)
        pltpu.sync_copy(vals_v, gp_hbm.at[pl.ds((b * 3 + d) * MK, MK)])

    @pl.when(u == 0)
    def _():
        do_coord(pxf_hbm, cxf_hbm, 0)

    @pl.when(u == 1)
    def _():
        do_coord(pyf_hbm, cyf_hbm, 1)

    @pl.when(u == 2)
    def _():
        do_coord(pzf_hbm, czf_hbm, 2)


def _run_gather(x, px, py, pz, cx, cy, cz, idx, nidx):
    f32 = jnp.float32
    MK = M * K
    mesh = plsc.VectorSubcoreMesh(core_axis_name="c", subcore_axis_name="s")
    kn = functools.partial(
        pl.kernel,
        out_type=(
            jax.ShapeDtypeStruct((B * C * MK,), f32),
            jax.ShapeDtypeStruct((B * C * M,), f32),
            jax.ShapeDtypeStruct((B * 3 * MK,), f32),
        ),
        mesh=mesh,
        compiler_params=pltpu.CompilerParams(needs_layout_passes=False),
        scratch_types=[
            pltpu.VMEM((MK,), jnp.int32),
            pltpu.VMEM((MK,), jnp.int32),
            pltpu.VMEM((MK,), f32),
            pltpu.VMEM((M,), jnp.int32),
            pltpu.VMEM((M,), jnp.int32),
            pltpu.VMEM((M,), f32),
            pltpu.VMEM((M,), f32),
            pltpu.SemaphoreType.DMA,
        ],
    )(_gather_body)
    return kn(x.reshape(-1), px.reshape(-1), py.reshape(-1), pz.reshape(-1),
              cx.reshape(-1), cy.reshape(-1), cz.reshape(-1),
              idx.reshape(-1), nidx)


def kernel(p, x):
    px = p[:, :, 0]
    py = p[:, :, 1]
    pz = p[:, :, 2]

    idx, cx, cy, cz, c2, p2 = _run_fps(px, py, pz)

    center_p = jnp.stack([cx, cy, cz], axis=-1)
    p_t = jnp.stack([px, py, pz], axis=1)
    d2 = _run_d2(center_p, p_t, c2.reshape(B, M, 1), p2.reshape(B, 1, N))
    nidx = _run_sel(d2)

    fjf, cxof, gpf = _run_gather(x, px, py, pz, cx, cy, cz, idx, nidx)

    grouped_p = gpf.reshape(B, 3, M, K)
    fj = fjf.reshape(B, C, M, K)
    center_x = cxof.reshape(B, C, M, 1)
    return (grouped_p, center_p, fj, center_x)


# speedup vs baseline: 101.1643x; 1.1027x over previous
import functools

import jax
import jax.numpy as jnp
from jax import lax
from jax.experimental import pallas as pl
from jax.experimental.pallas import tpu as pltpu
from jax.experimental.pallas import tpu_sc as plsc

B = 4
N = 16384
C = 64
M = 1024
K = 32

NUM_WORKERS = 32
M_PER_W = (B * M) // NUM_WORKERS
CH_PER_W = (B * C) // NUM_WORKERS

F_BIG = 1e30
I_BIG = 1 << 20


def _rne_bf16(x):
    b = lax.bitcast_convert_type(x, jnp.uint32)
    lsb = (b >> 16) & jnp.uint32(1)
    r = b + jnp.uint32(0x7FFF) + lsb
    return lax.bitcast_convert_type(r & jnp.uint32(0xFFFF0000), jnp.float32)


SL = 8 * B
NL = N // 8


def _fps_body(px_ref, py_ref, pz_ref,
              idx_ref, cx_ref, cy_ref, cz_ref, c2_ref, p2_ref,
              dists_ref):
    px = px_ref[...]
    py = py_ref[...]
    pz = pz_ref[...]

    p2_ref[...] = (px * px + py * py) + pz * pz

    iota = (lax.broadcasted_iota(jnp.int32, (SL, NL), 0) % 8) * NL \
        + lax.broadcasted_iota(jnp.int32, (SL, NL), 1)
    lane128 = lax.broadcasted_iota(jnp.int32, (B, 128), 1)

    dists_ref[...] = jnp.full((SL, NL), 1e10, jnp.float32)

    def brow(scalars):
        return jnp.concatenate([jnp.full((8, 1), s) for s in scalars], axis=0)

    def bcol(scalars, dtype):
        return jnp.concatenate(
            [jnp.full((1, 1), s, dtype) for s in scalars], axis=0)

    cx0s = [px[8 * b, 0] for b in range(B)]
    cy0s = [py[8 * b, 0] for b in range(B)]
    cz0s = [pz[8 * b, 0] for b in range(B)]

    col0 = lane128 == 0
    zf = jnp.zeros((B, 128), jnp.float32)
    zi = jnp.zeros((B, 128), jnp.int32)
    ccx0 = bcol(cx0s, jnp.float32)
    ccy0 = bcol(cy0s, jnp.float32)
    ccz0 = bcol(cz0s, jnp.float32)
    bufs0 = (zi,
             jnp.where(col0, ccx0, zf), jnp.where(col0, ccy0, zf),
             jnp.where(col0, ccz0, zf),
             jnp.where(col0, (ccx0 * ccx0 + ccy0 * ccy0) + ccz0 * ccz0, zf))

    def step(i, carry):
        cx, cy, cz, bidx, bcx, bcy, bcz, bc2 = carry
        d = ((px - cx) ** 2 + (py - cy) ** 2) + (pz - cz) ** 2
        dn = jnp.minimum(dists_ref[...], d)
        dists_ref[...] = dn
        m1 = jnp.max(dn, axis=1, keepdims=True)
        mxs = [jnp.max(m1[8 * b:8 * b + 8, 0:1]) for b in range(B)]
        eq = jnp.where(dn == brow(mxs), iota, N)
        n1 = jnp.min(eq, axis=1, keepdims=True)
        nxts = [jnp.min(n1[8 * b:8 * b + 8, 0:1]) for b in range(B)]
        sel = iota == brow(nxts)
        gx = jnp.max(jnp.where(sel, px, -1.0), axis=1, keepdims=True)
        gy = jnp.max(jnp.where(sel, py, -1.0), axis=1, keepdims=True)
        gz = jnp.max(jnp.where(sel, pz, -1.0), axis=1, keepdims=True)
        cxs = [jnp.max(gx[8 * b:8 * b + 8, 0:1]) for b in range(B)]
        cys = [jnp.max(gy[8 * b:8 * b + 8, 0:1]) for b in range(B)]
        czs = [jnp.max(gz[8 * b:8 * b + 8, 0:1]) for b in range(B)]
        ncx = bcol(cxs, jnp.float32)
        ncy = bcol(cys, jnp.float32)
        ncz = bcol(czs, jnp.float32)
        nxt = bcol(nxts, jnp.int32)
        at = lane128 == (i % 128)
        bidx = jnp.where(at, nxt, bidx)
        bcx = jnp.where(at, ncx, bcx)
        bcy = jnp.where(at, ncy, bcy)
        bcz = jnp.where(at, ncz, bcz)
        bc2 = jnp.where(at, (ncx * ncx + ncy * ncy) + ncz * ncz, bc2)

        @pl.when(i % 128 == 127)
        def _flush():
            off = pl.multiple_of((i // 128) * 128, 128)
            idx_ref[:, pl.ds(off, 128)] = bidx
            cx_ref[:, pl.ds(off, 128)] = bcx
            cy_ref[:, pl.ds(off, 128)] = bcy
            cz_ref[:, pl.ds(off, 128)] = bcz
            c2_ref[:, pl.ds(off, 128)] = bc2

        return (brow(cxs), brow(cys), brow(czs),
                bidx, bcx, bcy, bcz, bc2)

    lax.fori_loop(1, M, step, (brow(cx0s), brow(cy0s), brow(cz0s)) + bufs0)


def _run_fps(px, py, pz):
    f32 = jnp.float32
    outs = [
        jax.ShapeDtypeStruct((B, M), jnp.int32),
        jax.ShapeDtypeStruct((B, M), f32),
        jax.ShapeDtypeStruct((B, M), f32),
        jax.ShapeDtypeStruct((B, M), f32),
        jax.ShapeDtypeStruct((B, M), f32),
        jax.ShapeDtypeStruct((SL, NL), f32),
    ]
    return pl.pallas_call(
        _fps_body,
        out_shape=outs,
        scratch_shapes=[pltpu.VMEM((SL, NL), f32)],
    )(px.reshape(SL, NL), py.reshape(SL, NL), pz.reshape(SL, NL))


NBLK = 64
VPB = 16


MBLK = 128


def _d2_body(cp_ref, pt_ref, c2_ref, p2_ref, out_ref):
    cp = cp_ref[0]
    pt = pt_ref[0]
    ein = jnp.dot(cp, pt, preferred_element_type=jnp.float32)
    out_ref[0] = (c2_ref[0] - 2.0 * ein) + p2_ref[0]


def _run_d2(center_p, p_t, c2c, p2r):
    return pl.pallas_call(
        _d2_body,
        grid=(B, M // MBLK),
        in_specs=[
            pl.BlockSpec((1, MBLK, 3), lambda b, m: (b, m, 0)),
            pl.BlockSpec((1, 3, N), lambda b, m: (b, 0, 0)),
            pl.BlockSpec((1, MBLK, 1), lambda b, m: (b, m, 0)),
            pl.BlockSpec((1, 1, N), lambda b, m: (b, 0, 0)),
        ],
        out_specs=pl.BlockSpec((1, MBLK, N), lambda b, m: (b, m, 0)),
        out_shape=jax.ShapeDtypeStruct((B, M, N), jnp.float32),
    )(center_p, p_t, c2c, p2r)


def _sel_body(d2_hbm, nidx_hbm, rowa_v, rowb_v, res_v, sema, semb):
    wid = lax.axis_index("s") * 2 + lax.axis_index("c")
    wbase = wid * M_PER_W

    ii = lax.broadcasted_iota(jnp.int32, (16,), 0)

    def select32(d2ref, lvl, m):
        def per_round(r, carry):
            w0, w1, l0, l1, l2, l3 = carry
            t = jnp.min(jnp.minimum(jnp.minimum(l0, l1),
                                    jnp.minimum(l2, l3)))
            j0 = jnp.where(l0 == t, ii, I_BIG)
            j1 = jnp.where(l1 == t, ii + 16, I_BIG)
            j2 = jnp.where(l2 == t, ii + 32, I_BIG)
            j3 = jnp.where(l3 == t, ii + 48, I_BIG)
            j = jnp.min(jnp.minimum(jnp.minimum(j0, j1),
                                    jnp.minimum(j2, j3)))
            base = j * 256
            pv = jnp.full((16,), I_BIG, jnp.int32)
            for k in range(VPB):
                o = pl.multiple_of(base + k * 16, 16)
                dv = d2ref[pl.ds(o, 16)]
                pv = jnp.minimum(pv, jnp.where(dv == t, ii + k * 16, I_BIG))
            pos = jnp.min(pv)
            n = base + pos
            vo = pl.multiple_of(base + (pos // 16) * 16, 16)
            lane = pos % 16
            dv = d2ref[pl.ds(vo, 16)]
            d2ref[pl.ds(vo, 16)] = jnp.where(ii == lane, F_BIG, dv)
            bmin = jnp.full((16,), F_BIG, jnp.float32)
            for k in range(VPB):
                o = pl.multiple_of(base + k * 16, 16)
                bmin = jnp.minimum(bmin, d2ref[pl.ds(o, 16)])
            s2 = jnp.min(bmin)
            gsel = j // 16
            lsel = ii == (j % 16)
            l0 = jnp.where(lsel & (gsel == 0), s2, l0)
            l1 = jnp.where(lsel & (gsel == 1), s2, l1)
            l2 = jnp.where(lsel & (gsel == 2), s2, l2)
            l3 = jnp.where(lsel & (gsel == 3), s2, l3)
            w0 = jnp.where(ii == r, n, w0)
            w1 = jnp.where(ii == r - 16, n, w1)
            return (w0, w1, l0, l1, l2, l3)

        zero16 = jnp.zeros((16,), jnp.int32)
        w0, w1 = lax.fori_loop(0, K, per_round,
                               (zero16, zero16) + lvl)[:2]
        ro = pl.multiple_of(m * K, 16)
        res_v[pl.ds(ro, 16)] = w0
        res_v[pl.ds(ro + 16, 16)] = w1

    def lvlpass(rowref):
        lvl = []
        for g in range(4):
            def per_block(blk, la):
                jb = g * 16 + blk
                bma = jnp.full((16,), F_BIG, jnp.float32)
                for k in range(VPB):
                    o = pl.multiple_of(jb * 256 + k * 16, 16)
                    bma = jnp.minimum(bma, rowref[pl.ds(o, 16)])
                return jnp.where(ii == blk, jnp.min(bma), la)

            lvl.append(lax.fori_loop(
                0, 16, per_block, jnp.full((16,), F_BIG, jnp.float32)))
        return tuple(lvl)

    pltpu.async_copy(d2_hbm.at[wbase], rowa_v, sema)

    def per_pair(h, _):
        mca = 2 * h
        mcb = 2 * h + 1
        hb = pltpu.async_copy(d2_hbm.at[wbase + mcb], rowb_v, semb)
        pltpu.make_async_copy(d2_hbm.at[0], rowa_v, sema).wait()
        select32(rowa_v, lvlpass(rowa_v), mca)
        nxt = jnp.where(mca + 2 < M_PER_W, mca + 2, 0)
        pltpu.async_copy(d2_hbm.at[wbase + nxt], rowa_v, sema)
        hb.wait()
        select32(rowb_v, lvlpass(rowb_v), mcb)
        return 0

    lax.fori_loop(0, M_PER_W // 2, per_pair, 0)
    pltpu.make_async_copy(d2_hbm.at[0], rowa_v, sema).wait()
    pltpu.sync_copy(res_v, nidx_hbm.at[pl.ds(wid * (M_PER_W * K),
                                             M_PER_W * K)])


def _run_sel(d2):
    f32 = jnp.float32
    mesh = plsc.VectorSubcoreMesh(core_axis_name="c", subcore_axis_name="s")
    kn = functools.partial(
        pl.kernel,
        out_type=jax.ShapeDtypeStruct((B * M * K,), jnp.int32),
        mesh=mesh,
        compiler_params=pltpu.CompilerParams(needs_layout_passes=False),
        scratch_types=[
            pltpu.VMEM((N,), f32),
            pltpu.VMEM((N,), f32),
            pltpu.VMEM((M_PER_W * K,), jnp.int32),
            pltpu.SemaphoreType.DMA,
            pltpu.SemaphoreType.DMA,
        ],
    )(_sel_body)
    return kn(d2.reshape(B * M, N))


def _gather_body(xflat_hbm, pxf_hbm, pyf_hbm, pzf_hbm,
                 cxf_hbm, cyf_hbm, czf_hbm, idx_hbm, nidx_hbm,
                 fj_hbm, cxo_hbm, gp_hbm,
                 nidx_v, absidx_v, vals_v, idx1_v, cabs_v, cvals_v, cd_v,
                 sem):
    wid = lax.axis_index("s") * 2 + lax.axis_index("c")
    b = wid // 8
    u = wid % 8
    MK = M * K

    pltpu.sync_copy(nidx_hbm.at[pl.ds(b * MK, MK)], nidx_v)
    pltpu.sync_copy(idx_hbm.at[pl.ds(b * M, M)], idx1_v)

    def add_off(i, off):
        o = pl.multiple_of(i * 16, 16)
        absidx_v[pl.ds(o, 16)] = nidx_v[pl.ds(o, 16)] + off
        return off

    def cadd_off(i, off):
        o = pl.multiple_of(i * 16, 16)
        cabs_v[pl.ds(o, 16)] = idx1_v[pl.ds(o, 16)] + off
        return off

    for c8 in range(CH_PER_W):
        ch = b * C + u * CH_PER_W + c8
        off = ch * N
        lax.fori_loop(0, MK // 16, add_off, off)
        pltpu.async_copy(xflat_hbm.at[absidx_v], vals_v, sem).wait()
        pltpu.sync_copy(vals_v, fj_hbm.at[pl.ds(ch * MK, MK)])
        lax.fori_loop(0, M // 16, cadd_off, off)
        pltpu.async_copy(xflat_hbm.at[cabs_v], cvals_v, sem).wait()
        pltpu.sync_copy(cvals_v, cxo_hbm.at[pl.ds(ch * M, M)])

    def do_coord(plane_hbm, cent_hbm, d):
        lax.fori_loop(0, MK // 16, add_off, b * N)
        pltpu.async_copy(plane_hbm.at[absidx_v], vals_v, sem).wait()
        pltpu.sync_copy(cent_hbm.at[pl.ds(b * M, M)], cd_v)

        def sub(i, _):
            o = pl.multiple_of(i * 16, 16)
            gvec = jnp.full((16,), 0, jnp.int32) + i // 2
            cs = plsc.load_gather(cd_v, [gvec])
            vals_v[pl.ds(o, 16)] = vals_v[pl.ds(o, 16)] - cs
            return 0

        lax.fori_loop(0, MK // 16, sub, 0)
        pltpu.sync_copy(vals_v, gp_hbm.at[pl.ds((b * 3 + d) * MK, MK)])

    @pl.when(u == 0)
    def _():
        do_coord(pxf_hbm, cxf_hbm, 0)

    @pl.when(u == 1)
    def _():
        do_coord(pyf_hbm, cyf_hbm, 1)

    @pl.when(u == 2)
    def _():
        do_coord(pzf_hbm, czf_hbm, 2)


def _run_gather(x, px, py, pz, cx, cy, cz, idx, nidx):
    f32 = jnp.float32
    MK = M * K
    mesh = plsc.VectorSubcoreMesh(core_axis_name="c", subcore_axis_name="s")
    kn = functools.partial(
        pl.kernel,
        out_type=(
            jax.ShapeDtypeStruct((B * C * MK,), f32),
            jax.ShapeDtypeStruct((B * C * M,), f32),
            jax.ShapeDtypeStruct((B * 3 * MK,), f32),
        ),
        mesh=mesh,
        compiler_params=pltpu.CompilerParams(needs_layout_passes=False),
        scratch_types=[
            pltpu.VMEM((MK,), jnp.int32),
            pltpu.VMEM((MK,), jnp.int32),
            pltpu.VMEM((MK,), f32),
            pltpu.VMEM((M,), jnp.int32),
            pltpu.VMEM((M,), jnp.int32),
            pltpu.VMEM((M,), f32),
            pltpu.VMEM((M,), f32),
            pltpu.SemaphoreType.DMA,
        ],
    )(_gather_body)
    return kn(x.reshape(-1), px.reshape(-1), py.reshape(-1), pz.reshape(-1),
              cx.reshape(-1), cy.reshape(-1), cz.reshape(-1),
              idx.reshape(-1), nidx)


def kernel(p, x):
    px = p[:, :, 0]
    py = p[:, :, 1]
    pz = p[:, :, 2]

    idx, cx, cy, cz, c2, p2 = _run_fps(px, py, pz)

    center_p = jnp.stack([cx, cy, cz], axis=-1)
    p_t = jnp.stack([px, py, pz], axis=1)
    d2 = _run_d2(center_p, p_t, c2.reshape(B, M, 1), p2.reshape(B, 1, N))
    nidx = _run_sel(d2)

    fjf, cxof, gpf = _run_gather(x, px, py, pz, cx, cy, cz, idx, nidx)

    grouped_p = gpf.reshape(B, 3, M, K)
    fj = fjf.reshape(B, C, M, K)
    center_x = cxof.reshape(B, C, M, 1)
    return (grouped_p, center_p, fj, center_x)


# speedup vs baseline: 117.3500x; 1.1600x over previous
import functools

import jax
import jax.numpy as jnp
from jax import lax
from jax.experimental import pallas as pl
from jax.experimental.pallas import tpu as pltpu
from jax.experimental.pallas import tpu_sc as plsc

B = 4
N = 16384
C = 64
M = 1024
K = 32

NUM_WORKERS = 32
M_PER_W = (B * M) // NUM_WORKERS
CH_PER_W = (B * C) // NUM_WORKERS

F_BIG = 1e30
I_BIG = 1 << 20


def _rne_bf16(x):
    b = lax.bitcast_convert_type(x, jnp.uint32)
    lsb = (b >> 16) & jnp.uint32(1)
    r = b + jnp.uint32(0x7FFF) + lsb
    return lax.bitcast_convert_type(r & jnp.uint32(0xFFFF0000), jnp.float32)


SL = 8 * B
NL = N // 8


def _fps_body(px_ref, py_ref, pz_ref,
              idx_ref, cx_ref, cy_ref, cz_ref, c2_ref, p2_ref,
              dists_ref):
    px = px_ref[...]
    py = py_ref[...]
    pz = pz_ref[...]

    p2_ref[...] = (px * px + py * py) + pz * pz

    iota = (lax.broadcasted_iota(jnp.int32, (SL, NL), 0) % 8) * NL \
        + lax.broadcasted_iota(jnp.int32, (SL, NL), 1)
    lane128 = lax.broadcasted_iota(jnp.int32, (B, 128), 1)

    dists_ref[...] = jnp.full((SL, NL), 1e10, jnp.float32)

    def brow(scalars):
        return jnp.concatenate([jnp.full((8, 1), s) for s in scalars], axis=0)

    def bcol(scalars, dtype):
        return jnp.concatenate(
            [jnp.full((1, 1), s, dtype) for s in scalars], axis=0)

    cx0s = [px[8 * b, 0] for b in range(B)]
    cy0s = [py[8 * b, 0] for b in range(B)]
    cz0s = [pz[8 * b, 0] for b in range(B)]

    col0 = lane128 == 0
    zf = jnp.zeros((B, 128), jnp.float32)
    zi = jnp.zeros((B, 128), jnp.int32)
    ccx0 = bcol(cx0s, jnp.float32)
    ccy0 = bcol(cy0s, jnp.float32)
    ccz0 = bcol(cz0s, jnp.float32)
    bufs0 = (zi,
             jnp.where(col0, ccx0, zf), jnp.where(col0, ccy0, zf),
             jnp.where(col0, ccz0, zf),
             jnp.where(col0, (ccx0 * ccx0 + ccy0 * ccy0) + ccz0 * ccz0, zf))

    def step(i, carry):
        cx, cy, cz, bidx, bcx, bcy, bcz, bc2 = carry
        d = ((px - cx) ** 2 + (py - cy) ** 2) + (pz - cz) ** 2
        dn = jnp.minimum(dists_ref[...], d)
        dists_ref[...] = dn
        m1 = jnp.max(dn, axis=1, keepdims=True)
        mxs = [jnp.max(m1[8 * b:8 * b + 8, 0:1]) for b in range(B)]
        eq = jnp.where(dn == brow(mxs), iota, N)
        n1 = jnp.min(eq, axis=1, keepdims=True)
        nxts = [jnp.min(n1[8 * b:8 * b + 8, 0:1]) for b in range(B)]
        sel = iota == brow(nxts)
        gx = jnp.max(jnp.where(sel, px, -1.0), axis=1, keepdims=True)
        gy = jnp.max(jnp.where(sel, py, -1.0), axis=1, keepdims=True)
        gz = jnp.max(jnp.where(sel, pz, -1.0), axis=1, keepdims=True)
        cxs = [jnp.max(gx[8 * b:8 * b + 8, 0:1]) for b in range(B)]
        cys = [jnp.max(gy[8 * b:8 * b + 8, 0:1]) for b in range(B)]
        czs = [jnp.max(gz[8 * b:8 * b + 8, 0:1]) for b in range(B)]
        ncx = bcol(cxs, jnp.float32)
        ncy = bcol(cys, jnp.float32)
        ncz = bcol(czs, jnp.float32)
        nxt = bcol(nxts, jnp.int32)
        at = lane128 == (i % 128)
        bidx = jnp.where(at, nxt, bidx)
        bcx = jnp.where(at, ncx, bcx)
        bcy = jnp.where(at, ncy, bcy)
        bcz = jnp.where(at, ncz, bcz)
        bc2 = jnp.where(at, (ncx * ncx + ncy * ncy) + ncz * ncz, bc2)

        @pl.when(i % 128 == 127)
        def _flush():
            off = pl.multiple_of((i // 128) * 128, 128)
            idx_ref[:, pl.ds(off, 128)] = bidx
            cx_ref[:, pl.ds(off, 128)] = bcx
            cy_ref[:, pl.ds(off, 128)] = bcy
            cz_ref[:, pl.ds(off, 128)] = bcz
            c2_ref[:, pl.ds(off, 128)] = bc2

        return (brow(cxs), brow(cys), brow(czs),
                bidx, bcx, bcy, bcz, bc2)

    lax.fori_loop(1, M, step, (brow(cx0s), brow(cy0s), brow(cz0s)) + bufs0)


def _run_fps(px, py, pz):
    f32 = jnp.float32
    outs = [
        jax.ShapeDtypeStruct((B, M), jnp.int32),
        jax.ShapeDtypeStruct((B, M), f32),
        jax.ShapeDtypeStruct((B, M), f32),
        jax.ShapeDtypeStruct((B, M), f32),
        jax.ShapeDtypeStruct((B, M), f32),
        jax.ShapeDtypeStruct((SL, NL), f32),
    ]
    return pl.pallas_call(
        _fps_body,
        out_shape=outs,
        scratch_shapes=[pltpu.VMEM((SL, NL), f32)],
    )(px.reshape(SL, NL), py.reshape(SL, NL), pz.reshape(SL, NL))


NBLK = 64
VPB = 16


MBLK = 128


def _d2_body(cp_ref, pt_ref, c2_ref, p2_ref, out_ref):
    cp = cp_ref[0]
    pt = pt_ref[0]
    ein = jnp.dot(cp, pt, preferred_element_type=jnp.float32)
    out_ref[0] = (c2_ref[0] - 2.0 * ein) + p2_ref[0]


def _run_d2(center_p, p_t, c2c, p2r):
    return pl.pallas_call(
        _d2_body,
        grid=(B, M // MBLK),
        in_specs=[
            pl.BlockSpec((1, MBLK, 3), lambda b, m: (b, m, 0)),
            pl.BlockSpec((1, 3, N), lambda b, m: (b, 0, 0)),
            pl.BlockSpec((1, MBLK, 1), lambda b, m: (b, m, 0)),
            pl.BlockSpec((1, 1, N), lambda b, m: (b, 0, 0)),
        ],
        out_specs=pl.BlockSpec((1, MBLK, N), lambda b, m: (b, m, 0)),
        out_shape=jax.ShapeDtypeStruct((B, M, N), jnp.float32),
    )(center_p, p_t, c2c, p2r)


def _sel_body(d2_hbm, nidx_hbm, rowa_v, rowb_v, res_v, sema, semb):
    wid = lax.axis_index("s") * 2 + lax.axis_index("c")
    wbase = wid * M_PER_W

    ii = lax.broadcasted_iota(jnp.int32, (16,), 0)

    def select32(d2ref, lvl, m):
        def per_round(r, carry):
            w0, w1, l0, l1, l2, l3 = carry
            t = jnp.min(jnp.minimum(jnp.minimum(l0, l1),
                                    jnp.minimum(l2, l3)))
            j0 = jnp.where(l0 == t, ii, I_BIG)
            j1 = jnp.where(l1 == t, ii + 16, I_BIG)
            j2 = jnp.where(l2 == t, ii + 32, I_BIG)
            j3 = jnp.where(l3 == t, ii + 48, I_BIG)
            j = jnp.min(jnp.minimum(jnp.minimum(j0, j1),
                                    jnp.minimum(j2, j3)))
            base = j * 256
            pv = jnp.full((16,), I_BIG, jnp.int32)
            for k in range(VPB):
                o = pl.multiple_of(base + k * 16, 16)
                dv = d2ref[pl.ds(o, 16)]
                pv = jnp.minimum(pv, jnp.where(dv == t, ii + k * 16, I_BIG))
            pos = jnp.min(pv)
            n = base + pos
            vo = pl.multiple_of(base + (pos // 16) * 16, 16)
            lane = pos % 16
            dv = d2ref[pl.ds(vo, 16)]
            d2ref[pl.ds(vo, 16)] = jnp.where(ii == lane, F_BIG, dv)
            bmin = jnp.full((16,), F_BIG, jnp.float32)
            for k in range(VPB):
                o = pl.multiple_of(base + k * 16, 16)
                bmin = jnp.minimum(bmin, d2ref[pl.ds(o, 16)])
            s2 = jnp.min(bmin)
            gsel = j // 16
            lsel = ii == (j % 16)
            l0 = jnp.where(lsel & (gsel == 0), s2, l0)
            l1 = jnp.where(lsel & (gsel == 1), s2, l1)
            l2 = jnp.where(lsel & (gsel == 2), s2, l2)
            l3 = jnp.where(lsel & (gsel == 3), s2, l3)
            w0 = jnp.where(ii == r, n, w0)
            w1 = jnp.where(ii == r - 16, n, w1)
            return (w0, w1, l0, l1, l2, l3)

        zero16 = jnp.zeros((16,), jnp.int32)
        w0, w1 = lax.fori_loop(0, K, per_round,
                               (zero16, zero16) + lvl)[:2]
        ro = pl.multiple_of(m * K, 16)
        res_v[pl.ds(ro, 16)] = w0
        res_v[pl.ds(ro + 16, 16)] = w1

    def lvlpass(rowref):
        lvl = []
        for g in range(4):
            def per_block(blk, la):
                jb = g * 16 + blk
                bma = jnp.full((16,), F_BIG, jnp.float32)
                for k in range(VPB):
                    o = pl.multiple_of(jb * 256 + k * 16, 16)
                    bma = jnp.minimum(bma, rowref[pl.ds(o, 16)])
                return jnp.where(ii == blk, jnp.min(bma), la)

            lvl.append(lax.fori_loop(
                0, 16, per_block, jnp.full((16,), F_BIG, jnp.float32)))
        return tuple(lvl)

    pltpu.async_copy(d2_hbm.at[wbase], rowa_v, sema)

    def per_pair(h, _):
        mca = 2 * h
        mcb = 2 * h + 1
        hb = pltpu.async_copy(d2_hbm.at[wbase + mcb], rowb_v, semb)
        pltpu.make_async_copy(d2_hbm.at[0], rowa_v, sema).wait()
        select32(rowa_v, lvlpass(rowa_v), mca)
        nxt = jnp.where(mca + 2 < M_PER_W, mca + 2, 0)
        pltpu.async_copy(d2_hbm.at[wbase + nxt], rowa_v, sema)
        hb.wait()
        select32(rowb_v, lvlpass(rowb_v), mcb)
        return 0

    lax.fori_loop(0, M_PER_W // 2, per_pair, 0)
    pltpu.make_async_copy(d2_hbm.at[0], rowa_v, sema).wait()
    pltpu.sync_copy(res_v, nidx_hbm.at[pl.ds(wid * (M_PER_W * K),
                                             M_PER_W * K)])


def _run_sel(d2):
    f32 = jnp.float32
    mesh = plsc.VectorSubcoreMesh(core_axis_name="c", subcore_axis_name="s")
    kn = functools.partial(
        pl.kernel,
        out_type=jax.ShapeDtypeStruct((B * M * K,), jnp.int32),
        mesh=mesh,
        compiler_params=pltpu.CompilerParams(needs_layout_passes=False),
        scratch_types=[
            pltpu.VMEM((N,), f32),
            pltpu.VMEM((N,), f32),
            pltpu.VMEM((M_PER_W * K,), jnp.int32),
            pltpu.SemaphoreType.DMA,
            pltpu.SemaphoreType.DMA,
        ],
    )(_sel_body)
    return kn(d2.reshape(B * M, N))


def _gather_body(xflat_hbm, xt_hbm, pxf_hbm, pyf_hbm, pzf_hbm,
                 cxf_hbm, cyf_hbm, czf_hbm, idx_hbm, nidx_hbm,
                 fr_hbm, cxo_hbm, gp_hbm,
                 nidx_v, absidx_v, vals_v, rows_v,
                 idx1_v, cabs_v, cvals_v, cd_v,
                 sem):
    wid = lax.axis_index("s") * 2 + lax.axis_index("c")
    b = wid // 8
    u = wid % 8
    MK = M * K
    RPW = MK // 8
    RCH = 512

    pltpu.sync_copy(nidx_hbm.at[pl.ds(b * MK, MK)], nidx_v)
    pltpu.sync_copy(idx_hbm.at[pl.ds(b * M, M)], idx1_v)

    def cadd_off(i, off):
        o = pl.multiple_of(i * 16, 16)
        cabs_v[pl.ds(o, 16)] = idx1_v[pl.ds(o, 16)] + off
        return off

    def fj_chunk(q, _):
        lbase = pl.multiple_of(u * RPW + q * RCH, 16)
        def radd(i, _):
            o = pl.multiple_of(i * 16, 16)
            absidx_v[pl.ds(o, 16)] = nidx_v[pl.ds(lbase + o, 16)] + b * N
            return 0
        lax.fori_loop(0, RCH // 16, radd, 0)
        pltpu.async_copy(
            xt_hbm.at[absidx_v.at[pl.ds(0, RCH)]], rows_v, sem).wait()
        pltpu.sync_copy(rows_v,
                        fr_hbm.at[pl.ds(wid * RPW + q * RCH, RCH)])
        return 0

    lax.fori_loop(0, RPW // RCH, fj_chunk, 0)

    for c8 in range(CH_PER_W):
        ch = b * C + u * CH_PER_W + c8
        off = ch * N
        lax.fori_loop(0, M // 16, cadd_off, off)
        pltpu.async_copy(xflat_hbm.at[cabs_v], cvals_v, sem).wait()
        pltpu.sync_copy(cvals_v, cxo_hbm.at[pl.ds(ch * M, M)])

    GCH = 4096

    def do_coord(plane_hbm, cent_hbm, d):
        pltpu.sync_copy(cent_hbm.at[pl.ds(b * M, M)], cd_v)

        def gchunk(q, _):
            cb = pl.multiple_of(q * GCH, 16)

            def add(i, _):
                o = pl.multiple_of(i * 16, 16)
                absidx_v[pl.ds(o, 16)] = nidx_v[pl.ds(cb + o, 16)] + b * N
                return 0

            lax.fori_loop(0, GCH // 16, add, 0)
            pltpu.async_copy(plane_hbm.at[absidx_v], vals_v, sem).wait()

            def sub(i, _):
                o = pl.multiple_of(i * 16, 16)
                gvec = jnp.full((16,), 0, jnp.int32) + (cb // 32 + i // 2)
                cs = plsc.load_gather(cd_v, [gvec])
                vals_v[pl.ds(o, 16)] = vals_v[pl.ds(o, 16)] - cs
                return 0

            lax.fori_loop(0, GCH // 16, sub, 0)
            pltpu.sync_copy(
                vals_v, gp_hbm.at[pl.ds((b * 3 + d) * MK + q * GCH, GCH)])
            return 0

        lax.fori_loop(0, MK // GCH, gchunk, 0)

    @pl.when(u == 0)
    def _():
        do_coord(pxf_hbm, cxf_hbm, 0)

    @pl.when(u == 1)
    def _():
        do_coord(pyf_hbm, cyf_hbm, 1)

    @pl.when(u == 2)
    def _():
        do_coord(pzf_hbm, czf_hbm, 2)


def _run_gather(x, xt, px, py, pz, cx, cy, cz, idx, nidx):
    f32 = jnp.float32
    MK = M * K
    mesh = plsc.VectorSubcoreMesh(core_axis_name="c", subcore_axis_name="s")
    kn = functools.partial(
        pl.kernel,
        out_type=(
            jax.ShapeDtypeStruct((B * MK, 128), f32),
            jax.ShapeDtypeStruct((B * C * M,), f32),
            jax.ShapeDtypeStruct((B * 3 * MK,), f32),
        ),
        mesh=mesh,
        compiler_params=pltpu.CompilerParams(needs_layout_passes=False),
        scratch_types=[
            pltpu.VMEM((MK,), jnp.int32),
            pltpu.VMEM((4096,), jnp.int32),
            pltpu.VMEM((4096,), f32),
            pltpu.VMEM((512, 128), f32),
            pltpu.VMEM((M,), jnp.int32),
            pltpu.VMEM((M,), jnp.int32),
            pltpu.VMEM((M,), f32),
            pltpu.VMEM((M,), f32),
            pltpu.SemaphoreType.DMA,
        ],
    )(_gather_body)
    return kn(x.reshape(-1), xt,
              px.reshape(-1), py.reshape(-1), pz.reshape(-1),
              cx.reshape(-1), cy.reshape(-1), cz.reshape(-1),
              idx.reshape(-1), nidx)


def _xt_body(x_ref, o_ref):
    xt = jnp.transpose(x_ref[0], (1, 0))
    o_ref[0] = jnp.concatenate(
        [xt, jnp.zeros((1024, 128 - C), jnp.float32)], axis=1)


def _run_xt(x):
    return pl.pallas_call(
        _xt_body,
        grid=(B, N // 1024),
        in_specs=[pl.BlockSpec((1, C, 1024), lambda b, n: (b, 0, n))],
        out_specs=pl.BlockSpec((1, 1024, 128), lambda b, n: (b, n, 0)),
        out_shape=jax.ShapeDtypeStruct((B, N, 128), jnp.float32),
    )(x)


def _fjt_body(fr_ref, o_ref):
    t = jnp.transpose(fr_ref[0], (1, 0))
    o_ref[0] = t[:C, :]


def _run_fjt(fr):
    MK = M * K
    return pl.pallas_call(
        _fjt_body,
        grid=(B, MK // 1024),
        in_specs=[pl.BlockSpec((1, 1024, 128), lambda b, n: (b, n, 0))],
        out_specs=pl.BlockSpec((1, C, 1024), lambda b, n: (b, 0, n)),
        out_shape=jax.ShapeDtypeStruct((B, C, MK), jnp.float32),
    )(fr)


def kernel(p, x):
    px = p[:, :, 0]
    py = p[:, :, 1]
    pz = p[:, :, 2]

    idx, cx, cy, cz, c2, p2 = _run_fps(px, py, pz)

    center_p = jnp.stack([cx, cy, cz], axis=-1)
    p_t = jnp.stack([px, py, pz], axis=1)
    d2 = _run_d2(center_p, p_t, c2.reshape(B, M, 1), p2.reshape(B, 1, N))
    nidx = _run_sel(d2)

    xt = _run_xt(x).reshape(B * N, 128)
    frf, cxof, gpf = _run_gather(x, xt, px, py, pz, cx, cy, cz, idx, nidx)

    grouped_p = gpf.reshape(B, 3, M, K)
    fj = _run_fjt(frf.reshape(B, M * K, 128)).reshape(B, C, M, K)
    center_x = cxof.reshape(B, C, M, 1)
    return (grouped_p, center_p, fj, center_x)


# speedup vs baseline: 117.3681x; 1.0002x over previous
import functools

import jax
import jax.numpy as jnp
from jax import lax
from jax.experimental import pallas as pl
from jax.experimental.pallas import tpu as pltpu
from jax.experimental.pallas import tpu_sc as plsc

B = 4
N = 16384
C = 64
M = 1024
K = 32

NUM_WORKERS = 32
M_PER_W = (B * M) // NUM_WORKERS
CH_PER_W = (B * C) // NUM_WORKERS

F_BIG = 1e30
I_BIG = 1 << 20


SL = 8 * B
NL = N // 8


def _fps_body(px_ref, py_ref, pz_ref,
              idx_ref, cx_ref, cy_ref, cz_ref, c2_ref, p2_ref,
              dists_ref):
    px = px_ref[...]
    py = py_ref[...]
    pz = pz_ref[...]

    p2_ref[...] = (px * px + py * py) + pz * pz

    iota = (lax.broadcasted_iota(jnp.int32, (SL, NL), 0) % 8) * NL \
        + lax.broadcasted_iota(jnp.int32, (SL, NL), 1)
    lane128 = lax.broadcasted_iota(jnp.int32, (B, 128), 1)

    dists_ref[...] = jnp.full((SL, NL), 1e10, jnp.float32)

    def brow(scalars):
        return jnp.concatenate([jnp.full((8, 1), s) for s in scalars], axis=0)

    def bcol(scalars, dtype):
        return jnp.concatenate(
            [jnp.full((1, 1), s, dtype) for s in scalars], axis=0)

    cx0s = [px[8 * b, 0] for b in range(B)]
    cy0s = [py[8 * b, 0] for b in range(B)]
    cz0s = [pz[8 * b, 0] for b in range(B)]

    col0 = lane128 == 0
    zf = jnp.zeros((B, 128), jnp.float32)
    zi = jnp.zeros((B, 128), jnp.int32)
    ccx0 = bcol(cx0s, jnp.float32)
    ccy0 = bcol(cy0s, jnp.float32)
    ccz0 = bcol(cz0s, jnp.float32)
    bufs0 = (zi,
             jnp.where(col0, ccx0, zf), jnp.where(col0, ccy0, zf),
             jnp.where(col0, ccz0, zf),
             jnp.where(col0, (ccx0 * ccx0 + ccy0 * ccy0) + ccz0 * ccz0, zf))

    def step(i, carry):
        cx, cy, cz, bidx, bcx, bcy, bcz, bc2 = carry
        d = ((px - cx) ** 2 + (py - cy) ** 2) + (pz - cz) ** 2
        dn = jnp.minimum(dists_ref[...], d)
        dists_ref[...] = dn
        m1 = jnp.max(dn, axis=1, keepdims=True)
        mxs = [jnp.max(m1[8 * b:8 * b + 8, 0:1]) for b in range(B)]
        eq = jnp.where(dn == brow(mxs), iota, N)
        n1 = jnp.min(eq, axis=1, keepdims=True)
        nxts = [jnp.min(n1[8 * b:8 * b + 8, 0:1]) for b in range(B)]
        sel = iota == brow(nxts)
        gx = jnp.max(jnp.where(sel, px, -1.0), axis=1, keepdims=True)
        gy = jnp.max(jnp.where(sel, py, -1.0), axis=1, keepdims=True)
        gz = jnp.max(jnp.where(sel, pz, -1.0), axis=1, keepdims=True)
        cxs = [jnp.max(gx[8 * b:8 * b + 8, 0:1]) for b in range(B)]
        cys = [jnp.max(gy[8 * b:8 * b + 8, 0:1]) for b in range(B)]
        czs = [jnp.max(gz[8 * b:8 * b + 8, 0:1]) for b in range(B)]
        ncx = bcol(cxs, jnp.float32)
        ncy = bcol(cys, jnp.float32)
        ncz = bcol(czs, jnp.float32)
        nxt = bcol(nxts, jnp.int32)
        at = lane128 == (i % 128)
        bidx = jnp.where(at, nxt, bidx)
        bcx = jnp.where(at, ncx, bcx)
        bcy = jnp.where(at, ncy, bcy)
        bcz = jnp.where(at, ncz, bcz)
        bc2 = jnp.where(at, (ncx * ncx + ncy * ncy) + ncz * ncz, bc2)

        @pl.when(i % 128 == 127)
        def _flush():
            off = pl.multiple_of((i // 128) * 128, 128)
            idx_ref[:, pl.ds(off, 128)] = bidx
            cx_ref[:, pl.ds(off, 128)] = bcx
            cy_ref[:, pl.ds(off, 128)] = bcy
            cz_ref[:, pl.ds(off, 128)] = bcz
            c2_ref[:, pl.ds(off, 128)] = bc2

        return (brow(cxs), brow(cys), brow(czs),
                bidx, bcx, bcy, bcz, bc2)

    lax.fori_loop(1, M, step, (brow(cx0s), brow(cy0s), brow(cz0s)) + bufs0)


def _run_fps(px, py, pz):
    f32 = jnp.float32
    outs = [
        jax.ShapeDtypeStruct((B, M), jnp.int32),
        jax.ShapeDtypeStruct((B, M), f32),
        jax.ShapeDtypeStruct((B, M), f32),
        jax.ShapeDtypeStruct((B, M), f32),
        jax.ShapeDtypeStruct((B, M), f32),
        jax.ShapeDtypeStruct((SL, NL), f32),
    ]
    return pl.pallas_call(
        _fps_body,
        out_shape=outs,
        scratch_shapes=[pltpu.VMEM((SL, NL), f32)],
    )(px.reshape(SL, NL), py.reshape(SL, NL), pz.reshape(SL, NL))


NBLK = 64
VPB = 16


MBLK = 128


def _d2_body(cp_ref, pt_ref, c2_ref, p2_ref, out_ref):
    cp = cp_ref[0]
    pt = pt_ref[0]
    ein = jnp.dot(cp, pt, preferred_element_type=jnp.float32)
    out_ref[0] = (c2_ref[0] - 2.0 * ein) + p2_ref[0]


def _run_d2(center_p, p_t, c2c, p2r):
    return pl.pallas_call(
        _d2_body,
        grid=(B, M // MBLK),
        in_specs=[
            pl.BlockSpec((1, MBLK, 3), lambda b, m: (b, m, 0)),
            pl.BlockSpec((1, 3, N), lambda b, m: (b, 0, 0)),
            pl.BlockSpec((1, MBLK, 1), lambda b, m: (b, m, 0)),
            pl.BlockSpec((1, 1, N), lambda b, m: (b, 0, 0)),
        ],
        out_specs=pl.BlockSpec((1, MBLK, N), lambda b, m: (b, m, 0)),
        out_shape=jax.ShapeDtypeStruct((B, M, N), jnp.float32),
    )(center_p, p_t, c2c, p2r)


def _sel_body(d2_hbm, nidx_hbm, rowa_v, rowb_v, res_v, sema, semb):
    wid = lax.axis_index("s") * 2 + lax.axis_index("c")
    wbase = wid * M_PER_W

    ii = lax.broadcasted_iota(jnp.int32, (16,), 0)

    def select32(d2ref, lvl, m):
        def per_round(r, carry):
            w0, w1, l0, l1, l2, l3 = carry
            t = jnp.min(jnp.minimum(jnp.minimum(l0, l1),
                                    jnp.minimum(l2, l3)))
            j0 = jnp.where(l0 == t, ii, I_BIG)
            j1 = jnp.where(l1 == t, ii + 16, I_BIG)
            j2 = jnp.where(l2 == t, ii + 32, I_BIG)
            j3 = jnp.where(l3 == t, ii + 48, I_BIG)
            j = jnp.min(jnp.minimum(jnp.minimum(j0, j1),
                                    jnp.minimum(j2, j3)))
            base = j * 256
            pv = jnp.full((16,), I_BIG, jnp.int32)
            for k in range(VPB):
                o = pl.multiple_of(base + k * 16, 16)
                dv = d2ref[pl.ds(o, 16)]
                pv = jnp.minimum(pv, jnp.where(dv == t, ii + k * 16, I_BIG))
            pos = jnp.min(pv)
            n = base + pos
            vo = pl.multiple_of(base + (pos // 16) * 16, 16)
            lane = pos % 16
            dv = d2ref[pl.ds(vo, 16)]
            d2ref[pl.ds(vo, 16)] = jnp.where(ii == lane, F_BIG, dv)
            bmin = jnp.full((16,), F_BIG, jnp.float32)
            for k in range(VPB):
                o = pl.multiple_of(base + k * 16, 16)
                bmin = jnp.minimum(bmin, d2ref[pl.ds(o, 16)])
            s2 = jnp.min(bmin)
            gsel = j // 16
            lsel = ii == (j % 16)
            l0 = jnp.where(lsel & (gsel == 0), s2, l0)
            l1 = jnp.where(lsel & (gsel == 1), s2, l1)
            l2 = jnp.where(lsel & (gsel == 2), s2, l2)
            l3 = jnp.where(lsel & (gsel == 3), s2, l3)
            w0 = jnp.where(ii == r, n, w0)
            w1 = jnp.where(ii == r - 16, n, w1)
            return (w0, w1, l0, l1, l2, l3)

        zero16 = jnp.zeros((16,), jnp.int32)
        w0, w1 = lax.fori_loop(0, K, per_round,
                               (zero16, zero16) + lvl)[:2]
        ro = pl.multiple_of(m * K, 16)
        res_v[pl.ds(ro, 16)] = w0
        res_v[pl.ds(ro + 16, 16)] = w1

    def lvlpass(rowref):
        lvl = []
        for g in range(4):
            def per_block(blk, la):
                jb = g * 16 + blk
                bma = jnp.full((16,), F_BIG, jnp.float32)
                for k in range(VPB):
                    o = pl.multiple_of(jb * 256 + k * 16, 16)
                    bma = jnp.minimum(bma, rowref[pl.ds(o, 16)])
                return jnp.where(ii == blk, jnp.min(bma), la)

            lvl.append(lax.fori_loop(
                0, 16, per_block, jnp.full((16,), F_BIG, jnp.float32)))
        return tuple(lvl)

    pltpu.async_copy(d2_hbm.at[wbase], rowa_v, sema)

    def per_pair(h, _):
        mca = 2 * h
        mcb = 2 * h + 1
        hb = pltpu.async_copy(d2_hbm.at[wbase + mcb], rowb_v, semb)
        pltpu.make_async_copy(d2_hbm.at[0], rowa_v, sema).wait()
        select32(rowa_v, lvlpass(rowa_v), mca)
        nxt = jnp.where(mca + 2 < M_PER_W, mca + 2, 0)
        pltpu.async_copy(d2_hbm.at[wbase + nxt], rowa_v, sema)
        hb.wait()
        select32(rowb_v, lvlpass(rowb_v), mcb)
        return 0

    lax.fori_loop(0, M_PER_W // 2, per_pair, 0)
    pltpu.make_async_copy(d2_hbm.at[0], rowa_v, sema).wait()
    pltpu.sync_copy(res_v, nidx_hbm.at[pl.ds(wid * (M_PER_W * K),
                                             M_PER_W * K)])


def _run_sel(d2):
    f32 = jnp.float32
    mesh = plsc.VectorSubcoreMesh(core_axis_name="c", subcore_axis_name="s")
    kn = functools.partial(
        pl.kernel,
        out_type=jax.ShapeDtypeStruct((B * M * K,), jnp.int32),
        mesh=mesh,
        compiler_params=pltpu.CompilerParams(needs_layout_passes=False),
        scratch_types=[
            pltpu.VMEM((N,), f32),
            pltpu.VMEM((N,), f32),
            pltpu.VMEM((M_PER_W * K,), jnp.int32),
            pltpu.SemaphoreType.DMA,
            pltpu.SemaphoreType.DMA,
        ],
    )(_sel_body)
    return kn(d2.reshape(B * M, N))


def _gather_body(xflat_hbm, xt_hbm, pxf_hbm, pyf_hbm, pzf_hbm,
                 cxf_hbm, cyf_hbm, czf_hbm, idx_hbm, nidx_hbm,
                 fr_hbm, cxo_hbm, gp_hbm,
                 nidx_v, absidx_v, vals_v, rows_v,
                 idx1_v, cabs_v, cvals_v, cd_v,
                 sem):
    wid = lax.axis_index("s") * 2 + lax.axis_index("c")
    b = wid // 8
    u = wid % 8
    MK = M * K
    RPW = MK // 8
    RCH = 512

    pltpu.sync_copy(nidx_hbm.at[pl.ds(b * MK, MK)], nidx_v)
    pltpu.sync_copy(idx_hbm.at[pl.ds(b * M, M)], idx1_v)

    def cadd_off(i, off):
        o = pl.multiple_of(i * 16, 16)
        cabs_v[pl.ds(o, 16)] = idx1_v[pl.ds(o, 16)] + off
        return off

    def fj_chunk(q, _):
        lbase = pl.multiple_of(u * RPW + q * RCH, 16)
        def radd(i, _):
            o = pl.multiple_of(i * 16, 16)
            absidx_v[pl.ds(o, 16)] = nidx_v[pl.ds(lbase + o, 16)] + b * N
            return 0
        lax.fori_loop(0, RCH // 16, radd, 0)
        pltpu.async_copy(
            xt_hbm.at[absidx_v.at[pl.ds(0, RCH)]], rows_v, sem).wait()
        pltpu.sync_copy(rows_v,
                        fr_hbm.at[pl.ds(wid * RPW + q * RCH, RCH)])
        return 0

    lax.fori_loop(0, RPW // RCH, fj_chunk, 0)

    for c8 in range(CH_PER_W):
        ch = b * C + u * CH_PER_W + c8
        off = ch * N
        lax.fori_loop(0, M // 16, cadd_off, off)
        pltpu.async_copy(xflat_hbm.at[cabs_v], cvals_v, sem).wait()
        pltpu.sync_copy(cvals_v, cxo_hbm.at[pl.ds(ch * M, M)])

    GCH = 4096

    def do_coord(plane_hbm, cent_hbm, d):
        pltpu.sync_copy(cent_hbm.at[pl.ds(b * M, M)], cd_v)

        def gchunk(q, _):
            cb = pl.multiple_of(q * GCH, 16)

            def add(i, _):
                o = pl.multiple_of(i * 16, 16)
                absidx_v[pl.ds(o, 16)] = nidx_v[pl.ds(cb + o, 16)] + b * N
                return 0

            lax.fori_loop(0, GCH // 16, add, 0)
            pltpu.async_copy(plane_hbm.at[absidx_v], vals_v, sem).wait()

            def sub(i, _):
                o = pl.multiple_of(i * 16, 16)
                gvec = jnp.full((16,), 0, jnp.int32) + (cb // 32 + i // 2)
                cs = plsc.load_gather(cd_v, [gvec])
                vals_v[pl.ds(o, 16)] = vals_v[pl.ds(o, 16)] - cs
                return 0

            lax.fori_loop(0, GCH // 16, sub, 0)
            pltpu.sync_copy(
                vals_v, gp_hbm.at[pl.ds((b * 3 + d) * MK + q * GCH, GCH)])
            return 0

        lax.fori_loop(0, MK // GCH, gchunk, 0)

    @pl.when(u == 0)
    def _():
        do_coord(pxf_hbm, cxf_hbm, 0)

    @pl.when(u == 1)
    def _():
        do_coord(pyf_hbm, cyf_hbm, 1)

    @pl.when(u == 2)
    def _():
        do_coord(pzf_hbm, czf_hbm, 2)


def _run_gather(x, xt, px, py, pz, cx, cy, cz, idx, nidx):
    f32 = jnp.float32
    MK = M * K
    mesh = plsc.VectorSubcoreMesh(core_axis_name="c", subcore_axis_name="s")
    kn = functools.partial(
        pl.kernel,
        out_type=(
            jax.ShapeDtypeStruct((B * MK, 128), f32),
            jax.ShapeDtypeStruct((B * C * M,), f32),
            jax.ShapeDtypeStruct((B * 3 * MK,), f32),
        ),
        mesh=mesh,
        compiler_params=pltpu.CompilerParams(needs_layout_passes=False),
        scratch_types=[
            pltpu.VMEM((MK,), jnp.int32),
            pltpu.VMEM((4096,), jnp.int32),
            pltpu.VMEM((4096,), f32),
            pltpu.VMEM((512, 128), f32),
            pltpu.VMEM((M,), jnp.int32),
            pltpu.VMEM((M,), jnp.int32),
            pltpu.VMEM((M,), f32),
            pltpu.VMEM((M,), f32),
            pltpu.SemaphoreType.DMA,
        ],
    )(_gather_body)
    return kn(x.reshape(-1), xt,
              px.reshape(-1), py.reshape(-1), pz.reshape(-1),
              cx.reshape(-1), cy.reshape(-1), cz.reshape(-1),
              idx.reshape(-1), nidx)


def _xt_body(x_ref, o_ref):
    xt = jnp.transpose(x_ref[0], (1, 0))
    o_ref[0] = jnp.concatenate(
        [xt, jnp.zeros((1024, 128 - C), jnp.float32)], axis=1)


def _run_xt(x):
    return pl.pallas_call(
        _xt_body,
        grid=(B, N // 1024),
        in_specs=[pl.BlockSpec((1, C, 1024), lambda b, n: (b, 0, n))],
        out_specs=pl.BlockSpec((1, 1024, 128), lambda b, n: (b, n, 0)),
        out_shape=jax.ShapeDtypeStruct((B, N, 128), jnp.float32),
    )(x)


def _fjt_body(fr_ref, o_ref):
    t = jnp.transpose(fr_ref[0], (1, 0))
    o_ref[0] = t[:C, :]


def _run_fjt(fr):
    MK = M * K
    return pl.pallas_call(
        _fjt_body,
        grid=(B, MK // 1024),
        in_specs=[pl.BlockSpec((1, 1024, 128), lambda b, n: (b, n, 0))],
        out_specs=pl.BlockSpec((1, C, 1024), lambda b, n: (b, 0, n)),
        out_shape=jax.ShapeDtypeStruct((B, C, MK), jnp.float32),
    )(fr)


def kernel(p, x):
    px = p[:, :, 0]
    py = p[:, :, 1]
    pz = p[:, :, 2]

    idx, cx, cy, cz, c2, p2 = _run_fps(px, py, pz)

    center_p = jnp.stack([cx, cy, cz], axis=-1)
    p_t = jnp.stack([px, py, pz], axis=1)
    d2 = _run_d2(center_p, p_t, c2.reshape(B, M, 1), p2.reshape(B, 1, N))
    nidx = _run_sel(d2)

    xt = _run_xt(x).reshape(B * N, 128)
    frf, cxof, gpf = _run_gather(x, xt, px, py, pz, cx, cy, cz, idx, nidx)

    grouped_p = gpf.reshape(B, 3, M, K)
    fj = _run_fjt(frf.reshape(B, M * K, 128)).reshape(B, C, M, K)
    center_x = cxof.reshape(B, C, M, 1)
    return (grouped_p, center_p, fj, center_x)
